# Initial kernel scaffold; baseline (speedup 1.0000x reference)
import jax
import jax.numpy as jnp
from jax.experimental import pallas as pl


def kernel(x, edge_index, edge_attr, params):
    raise NotImplementedError("write your pallas kernel here")


# speedup vs baseline: 1.0243x; 1.0243x over previous
import functools

import jax
import jax.numpy as jnp
from jax import lax
from jax.experimental import pallas as pl
from jax.experimental.pallas import tpu as pltpu


def _mp_msg_body(u_ref, hs_ref, w2r_ref, b2r_ref, out_ref):
    u = u_ref[...]
    hs = hs_ref[...]
    acc = jnp.dot(hs, b2r_ref[...], preferred_element_type=jnp.float32)
    for k in range(u.shape[1]):
        acc += u[:, k:k + 1] * jnp.dot(hs, w2r_ref[k],
                                       preferred_element_type=jnp.float32)
    out_ref[...] = acc


def _mp_messages(u, hs, w2r, b2r):
    e, d = hs.shape
    kk = u.shape[1]
    eb = 2048
    return pl.pallas_call(
        _mp_msg_body,
        grid=(e // eb,),
        in_specs=[
            pl.BlockSpec((eb, kk), lambda i: (i, 0)),
            pl.BlockSpec((eb, d), lambda i: (i, 0)),
            pl.BlockSpec((kk, d, d), lambda i: (0, 0, 0)),
            pl.BlockSpec((d, d), lambda i: (0, 0)),
        ],
        out_specs=pl.BlockSpec((eb, d), lambda i: (i, 0)),
        out_shape=jax.ShapeDtypeStruct((e, d), jnp.float32),
    )(u, hs, w2r, b2r)


def _edge_softmax(scores, dst, n):
    mx = jax.ops.segment_max(scores, dst, num_segments=n)
    mx = jnp.where(jnp.isfinite(mx), mx, 0.0)
    ex = jnp.exp(scores - mx[dst])
    sm = jax.ops.segment_sum(ex, dst, num_segments=n)
    return ex / (sm[dst] + 1e-16)


def _gru_cell(xv, h, Wi, Wh, bi, bh):
    gi = xv @ Wi + bi
    gh = h @ Wh + bh
    ir, iz, inn = jnp.split(gi, 3, axis=-1)
    hr, hz, hn = jnp.split(gh, 3, axis=-1)
    r = jax.nn.sigmoid(ir + hr)
    z = jax.nn.sigmoid(iz + hz)
    nn_ = jnp.tanh(inn + r * hn)
    return (1.0 - z) * nn_ + z * h


def _gat_layer(h, src, dst, W, al, ar, heads, dh):
    n = h.shape[0]
    z = (h @ W).reshape(n, heads, dh)
    el = jnp.sum(z * al[None], axis=-1)
    er = jnp.sum(z * ar[None], axis=-1)
    e = jax.nn.leaky_relu(el[src] + er[dst], 0.2)
    a = _edge_softmax(e, dst, n)
    out = jax.ops.segment_sum(z[src] * a[:, :, None], dst, num_segments=n)
    return out.mean(axis=1)


def _gatt_layer(h, src, dst, W, b, aw, ab):
    n = h.shape[0]
    t = h @ W + b
    s = jnp.concatenate([t[src], t[dst]], axis=-1) @ aw
    s = s.squeeze(-1) + ab[0]
    s = jax.nn.leaky_relu(s, 0.2)
    a = _edge_softmax(s, dst, n)
    return jax.ops.segment_sum(t[src] * a[:, None], dst, num_segments=n)


def kernel(x, edge_index, edge_attr, params):
    p = params
    n = x.shape[0]
    d_in = x.shape[1]
    src = edge_index[0]
    dst = edge_index[1]

    heads = p['gat1_al'].shape[0]
    hid = p['gat1_al'].shape[1]
    d_out = p['gat3_al'].shape[1]

    h = jax.nn.relu(x @ p['proj_W'] + p['proj_b'])
    hidden = h
    u = jax.nn.relu(edge_attr @ p['enet_W1'] + p['enet_b1'])
    kk = u.shape[1]
    w2r = p['enet_W2'].reshape(kk, d_in, d_in)
    b2r = p['enet_b2'].reshape(d_in, d_in)
    for _ in range(3):
        hs = h[src]
        msg = _mp_messages(u, hs, w2r, b2r)
        agg = jax.ops.segment_sum(msg, dst, num_segments=n)
        m = jax.nn.relu(agg)
        hidden = _gru_cell(m, hidden, p['gru_Wi'], p['gru_Wh'],
                           p['gru_bi'], p['gru_bh'])
        h = hidden
    feats = h

    h = jax.nn.elu(_gat_layer(feats, src, dst, p['gat1_W'], p['gat1_al'],
                              p['gat1_ar'], heads, hid))
    h = jax.nn.elu(_gat_layer(h, src, dst, p['gat2_W'], p['gat2_al'],
                              p['gat2_ar'], heads, hid))
    h = _gat_layer(h, src, dst, p['gat3_W'], p['gat3_al'],
                   p['gat3_ar'], heads, d_out)

    h = _gatt_layer(h, src, dst, p['ga1_W'], p['ga1_b'], p['ga1_aw'], p['ga1_ab'])
    h = _gatt_layer(h, src, dst, p['ga2_W'], p['ga2_b'], p['ga2_aw'], p['ga2_ab'])
    h = _gatt_layer(h, src, dst, p['ga3_W'], p['ga3_b'], p['ga3_aw'], p['ga3_ab'])

    g_feats = jnp.sum(h, axis=0, keepdims=True)
    for _ in range(3):
        ctx = jnp.concatenate([jnp.broadcast_to(g_feats, h.shape), h], axis=-1)
        logits = jax.nn.leaky_relu(ctx @ p['ro_attW'] + p['ro_attb'], 0.01)
        a = jax.nn.softmax(logits, axis=0)
        hv = h @ p['ro_pW'] + p['ro_pb']
        context = jax.nn.elu(jnp.sum(a * hv, axis=0, keepdims=True))
        g_feats = _gru_cell(context, g_feats, p['ro_gWi'], p['ro_gWh'],
                            p['ro_gbi'], p['ro_gbh'])
    return g_feats


# speedup vs baseline: 5.6758x; 5.5414x over previous
import jax
import jax.numpy as jnp
from jax import lax
from jax.experimental import pallas as pl
from jax.experimental.pallas import tpu as pltpu
from jax.experimental.pallas import tpu_sc as plsc

F32 = jnp.float32
NC, NS = 2, 16
NW = NC * NS
BIG = 1e9
EPS = 1e-16


def _sc_mesh():
    return plsc.VectorSubcoreMesh(core_axis_name="c", subcore_axis_name="s")


def _pad_rows(w, rows):
    return jnp.concatenate(
        [w, jnp.zeros((rows - w.shape[0],) + w.shape[1:], F32)], axis=0)


def _sc_gather(tables, idxs):
    e = idxs[0].shape[0]
    ns = len(tables)
    epw = e // NW
    widths = [int(t.shape[1]) for t in tables]
    chs = [128 if w <= 256 else 64 for w in widths]
    out_type = tuple(jax.ShapeDtypeStruct((e, w), F32) for w in widths)
    scratch = []
    for w, ch in zip(widths, chs):
        scratch.append(pltpu.VMEM((ch,), jnp.int32))
        scratch.append(pltpu.VMEM((ch, w), F32))
    scratch.append(pltpu.SemaphoreType.DMA)

    def body(*refs):
        tabs = refs[:ns]
        idr = refs[ns:2 * ns]
        outs = refs[2 * ns:3 * ns]
        scr = refs[3 * ns:]
        sem = scr[-1]
        wid = lax.axis_index("s") * NC + lax.axis_index("c")
        base = wid * epw
        for i in range(ns):
            iv, rv = scr[2 * i], scr[2 * i + 1]
            ch = chs[i]
            for k in range(epw // ch):
                off = pl.multiple_of(base + k * ch, 8)
                pltpu.sync_copy(idr[i].at[pl.ds(off, ch)], iv)
                pltpu.async_copy(tabs[i].at[iv], rv, sem).wait()
                pltpu.sync_copy(rv, outs[i].at[pl.ds(off, ch)])

    return pl.kernel(body, out_type=out_type, mesh=_sc_mesh(),
                     scratch_types=tuple(scratch))(*tables, *idxs)


def _sc_scatter(payloads, dst_sel_flat, n):
    e = payloads[0].shape[0]
    ns = len(payloads)
    nq = n // 4
    acc_rows = nq + 128
    stripe = acc_rows // NS
    orows = nq // NS
    ept = e // NS
    ch = 128
    widths = [int(p.shape[1]) for p in payloads]
    out_type = tuple(jax.ShapeDtypeStruct((n, w), F32) for w in widths)
    scratch = [pltpu.VMEM((ch,), jnp.int32)]
    for w in widths:
        scratch.append(pltpu.VMEM((8, w), F32))
        scratch.append(pltpu.VMEM((ch, w), F32))
        scratch.append(pltpu.VMEM_SHARED((acc_rows, w), F32))
    scratch.append(pltpu.SemaphoreType.DMA)

    def body(dsr, *refs):
        pays = refs[:ns]
        outs = refs[ns:2 * ns]
        scr = refs[2 * ns:]
        iv = scr[0]
        sem = scr[-1]
        c = lax.axis_index("c")
        s = lax.axis_index("s")
        for q in range(2):
            for i in range(ns):
                zb, acc = scr[1 + 3 * i], scr[3 + 3 * i]
                if q == 0:
                    for r in range(8):
                        for j in range(widths[i] // 16):
                            zb[r, pl.ds(j * 16, 16)] = jnp.zeros((16,), F32)
                for j in range(stripe // 8):
                    zo = pl.multiple_of(s * stripe + j * 8, 8)
                    pltpu.sync_copy(zb, acc.at[pl.ds(zo, 8)])
            plsc.subcore_barrier()
            for k in range(ept // ch):
                ioff = pl.multiple_of((2 * c + q) * e + s * ept + k * ch, 8)
                eoff = pl.multiple_of(s * ept + k * ch, 8)
                pltpu.sync_copy(dsr.at[pl.ds(ioff, ch)], iv)
                for i in range(ns):
                    pv, acc = scr[2 + 3 * i], scr[3 + 3 * i]
                    pltpu.sync_copy(pays[i].at[pl.ds(eoff, ch)], pv)
                    pltpu.async_copy(pv, acc.at[iv], sem, add=True).wait()
            plsc.subcore_barrier()
            so = pl.multiple_of(s * orows, 8)
            oo = pl.multiple_of((2 * c + q) * nq + s * orows, 8)
            for i in range(ns):
                acc = scr[3 + 3 * i]
                pltpu.sync_copy(acc.at[pl.ds(so, orows)],
                                outs[i].at[pl.ds(oo, orows)])
            plsc.subcore_barrier()

    return pl.kernel(body, out_type=out_type, mesh=_sc_mesh(),
                     scratch_types=tuple(scratch))(dst_sel_flat, *payloads)


def _tc_dstsel(dst2d, n):
    e = dst2d.shape[1]
    nq = n // 4
    eb = 2048

    def body(d_ref, o_ref):
        d = d_ref[...]
        sels = []
        for q in range(4):
            dq = d - q * nq
            sels.append(jnp.where((dq >= 0) & (dq < nq), dq, nq))
        o_ref[...] = jnp.concatenate(sels, axis=0)

    return pl.pallas_call(
        body, grid=(e // eb,),
        in_specs=[pl.BlockSpec((1, eb), lambda i: (0, i))],
        out_specs=pl.BlockSpec((4, eb), lambda i: (0, i)),
        out_shape=jax.ShapeDtypeStruct((4, e), jnp.int32))(dst2d)


def _pre_act(xb, pre, d=None):
    if d is not None:
        xb = xb / (d[:, 0:1] + EPS)
    if pre == "relu":
        xb = jnp.maximum(xb, 0.0)
    elif pre == "elu":
        xb = jnp.where(xb > 0, xb, jnp.exp(jnp.minimum(xb, 0.0)) - 1.0)
    return xb


def _tc_linear(x, w, b, act="none", out_w=None, nb=1024):
    n, kdim = x.shape
    m = w.shape[-1]
    mo = m if out_w is None else out_w

    def body(x_ref, w_ref, b_ref, out_ref):
        y = jnp.dot(x_ref[...], w_ref[...], preferred_element_type=F32)
        y = y + b_ref[...]
        if act == "relu":
            y = jnp.maximum(y, 0.0)
        if mo > m:
            y = jnp.concatenate(
                [y, jnp.zeros((y.shape[0], mo - m), F32)], axis=1)
        out_ref[...] = y

    return pl.pallas_call(
        body, grid=(n // nb,),
        in_specs=[pl.BlockSpec((nb, kdim), lambda i: (i, 0)),
                  pl.BlockSpec((kdim, m), lambda i: (0, 0)),
                  pl.BlockSpec((1, m), lambda i: (0, 0))],
        out_specs=pl.BlockSpec((nb, mo), lambda i: (i, 0)),
        out_shape=jax.ShapeDtypeStruct((n, mo), F32))(x, w, b.reshape(1, m))


def _mp_msg_body(u_ref, hs_ref, w2r_ref, b2r_ref, out_ref):
    u = u_ref[...]
    hs = hs_ref[...]
    d = b2r_ref.shape[1]
    acc = jnp.dot(hs, b2r_ref[...], preferred_element_type=F32)
    for k in range(u.shape[1]):
        acc += u[:, k:k + 1] * jnp.dot(hs, w2r_ref[k],
                                       preferred_element_type=F32)
    acc = jnp.concatenate([acc, jnp.zeros((acc.shape[0], 128 - d), F32)],
                          axis=1)
    out_ref[...] = acc


def _mp_messages(u, hs, w2r, b2r):
    e = hs.shape[0]
    d = b2r.shape[1]
    kk = u.shape[1]
    eb = 2048
    return pl.pallas_call(
        _mp_msg_body,
        grid=(e // eb,),
        in_specs=[
            pl.BlockSpec((eb, kk), lambda i: (i, 0)),
            pl.BlockSpec((eb, 128), lambda i: (i, 0)),
            pl.BlockSpec((kk, 128, d), lambda i: (0, 0, 0)),
            pl.BlockSpec((128, d), lambda i: (0, 0)),
        ],
        out_specs=pl.BlockSpec((eb, 128), lambda i: (i, 0)),
        out_shape=jax.ShapeDtypeStruct((e, 128), F32))(u, hs, w2r, b2r)


def _tc_gru(agg, hidden, wi, wh, bi, bh, d, nb=1024):
    n = hidden.shape[0]

    def body(a_ref, h_ref, wi_ref, wh_ref, bi_ref, bh_ref, out_ref):
        m = jnp.maximum(a_ref[...], 0.0)
        hp = h_ref[...]
        gi = jnp.dot(m, wi_ref[...], preferred_element_type=F32) + bi_ref[...]
        gh = jnp.dot(hp, wh_ref[...], preferred_element_type=F32) + bh_ref[...]
        r = jax.nn.sigmoid(gi[:, 0:d] + gh[:, 0:d])
        z = jax.nn.sigmoid(gi[:, d:2 * d] + gh[:, d:2 * d])
        nn_ = jnp.tanh(gi[:, 2 * d:3 * d] + r * gh[:, 2 * d:3 * d])
        y = (1.0 - z) * nn_ + z * hp[:, 0:d]
        out_ref[...] = jnp.concatenate(
            [y, jnp.zeros((y.shape[0], 128 - d), F32)], axis=1)

    return pl.pallas_call(
        body, grid=(n // nb,),
        in_specs=[
            pl.BlockSpec((nb, 128), lambda i: (i, 0)),
            pl.BlockSpec((nb, 128), lambda i: (i, 0)),
            pl.BlockSpec((128, 3 * d), lambda i: (0, 0)),
            pl.BlockSpec((128, 3 * d), lambda i: (0, 0)),
            pl.BlockSpec((1, 3 * d), lambda i: (0, 0)),
            pl.BlockSpec((1, 3 * d), lambda i: (0, 0)),
        ],
        out_specs=pl.BlockSpec((nb, 128), lambda i: (i, 0)),
        out_shape=jax.ShapeDtypeStruct((n, 128), F32))(
            agg, hidden, wi, wh, bi.reshape(1, 3 * d), bh.reshape(1, 3 * d))


def _tc_gat_z(x, w, al, ar, heads, dh, pre="none", nb=1024):
    n, kdim = x.shape
    m = heads * dh

    def body(x_ref, w_ref, al_ref, ar_ref, z_ref, tab_ref, mx_ref):
        i = pl.program_id(0)
        xb = _pre_act(x_ref[...], pre)
        z = jnp.dot(xb, w_ref[...], preferred_element_type=F32)
        z_ref[...] = z
        els, ers = [], []
        for h in range(heads):
            zh = z[:, h * dh:(h + 1) * dh]
            els.append(jnp.sum(zh * al_ref[h:h + 1, :], axis=1, keepdims=True))
            ers.append(jnp.sum(zh * ar_ref[h:h + 1, :], axis=1, keepdims=True))
        el = jnp.concatenate(els, axis=1)
        er = jnp.concatenate(ers, axis=1)
        pad = jnp.zeros((xb.shape[0], 64 - heads), F32)
        tab_ref[...] = jnp.concatenate([el, pad, er, pad], axis=1)
        cur = jnp.concatenate(
            [jnp.max(el, axis=0, keepdims=True),
             jnp.max(er, axis=0, keepdims=True),
             jnp.full((1, 16 - 2 * heads), -BIG, F32)], axis=1)

        @pl.when(i == 0)
        def _():
            mx_ref[...] = cur

        @pl.when(i > 0)
        def _():
            mx_ref[...] = jnp.maximum(mx_ref[...], cur)

    return pl.pallas_call(
        body, grid=(n // nb,),
        in_specs=[
            pl.BlockSpec((nb, kdim), lambda i: (i, 0)),
            pl.BlockSpec((kdim, m), lambda i: (0, 0)),
            pl.BlockSpec((heads, dh), lambda i: (0, 0)),
            pl.BlockSpec((heads, dh), lambda i: (0, 0)),
        ],
        out_specs=[
            pl.BlockSpec((nb, m), lambda i: (i, 0)),
            pl.BlockSpec((nb, 128), lambda i: (i, 0)),
            pl.BlockSpec((1, 16), lambda i: (0, 0)),
        ],
        out_shape=[
            jax.ShapeDtypeStruct((n, m), F32),
            jax.ShapeDtypeStruct((n, 128), F32),
            jax.ShapeDtypeStruct((1, 16), F32),
        ])(x, w, al, ar)


def _tc_gat_ex(a, b, mx, heads, eb=2048):
    e = a.shape[0]

    def body(a_ref, b_ref, mx_ref, out_ref):
        m = mx_ref[...]
        s = m[:, 0:heads] + m[:, heads:2 * heads]
        s = jnp.maximum(s, 0.2 * s)
        ee = a_ref[..., 0:heads] + b_ref[..., 64:64 + heads]
        ee = jnp.maximum(ee, 0.2 * ee)
        ex = jnp.exp(ee - s)
        out_ref[...] = jnp.concatenate(
            [ex, jnp.zeros((ex.shape[0], 128 - heads), F32)], axis=1)

    return pl.pallas_call(
        body, grid=(e // eb,),
        in_specs=[
            pl.BlockSpec((eb, 128), lambda i: (i, 0)),
            pl.BlockSpec((eb, 128), lambda i: (i, 0)),
            pl.BlockSpec((1, 16), lambda i: (0, 0)),
        ],
        out_specs=pl.BlockSpec((eb, 128), lambda i: (i, 0)),
        out_shape=jax.ShapeDtypeStruct((e, 128), F32))(a, b, mx)


def _tc_binv(den, nb=1024):
    n = den.shape[0]

    def body(d_ref, o_ref):
        o_ref[...] = 1.0 / (d_ref[...] + EPS)

    return pl.pallas_call(
        body, grid=(n // nb,),
        in_specs=[pl.BlockSpec((nb, 128), lambda i: (i, 0))],
        out_specs=pl.BlockSpec((nb, 128), lambda i: (i, 0)),
        out_shape=jax.ShapeDtypeStruct((n, 128), F32))(den)


def _tc_gat_pay(zg, ex, bg, heads, dh, eb=2048):
    e, m = zg.shape
    mo = max(dh, 128)
    nch = mo // 128

    def body(z_ref, e_ref, b_ref, *outs):
        w = e_ref[..., 0:heads] * b_ref[..., 0:heads] * (1.0 / heads)
        z = z_ref[...]
        acc = w[:, 0:1] * z[:, 0:dh]
        for h in range(1, heads):
            acc += w[:, h:h + 1] * z[:, h * dh:(h + 1) * dh]
        if mo > dh:
            acc = jnp.concatenate(
                [acc, jnp.zeros((acc.shape[0], mo - dh), F32)], axis=1)
        for j in range(nch):
            outs[j][...] = acc[:, j * 128:(j + 1) * 128]

    return pl.pallas_call(
        body, grid=(e // eb,),
        in_specs=[
            pl.BlockSpec((eb, m), lambda i: (i, 0)),
            pl.BlockSpec((eb, 128), lambda i: (i, 0)),
            pl.BlockSpec((eb, 128), lambda i: (i, 0)),
        ],
        out_specs=[pl.BlockSpec((eb, 128), lambda i: (i, 0))] * nch,
        out_shape=[jax.ShapeDtypeStruct((e, 128), F32)] * nch)(zg, ex, bg)


def _tc_ga_t(x, w, b, awl, awr, ab, pre="none", den=None, nb=1024):
    n, kdim = x.shape
    m = w.shape[1]
    mo = max(m, 128)
    ins = [x, w, b.reshape(1, m), awl, awr, ab.reshape(1, 1)]
    specs = [pl.BlockSpec((nb, kdim), lambda i: (i, 0)),
             pl.BlockSpec((kdim, m), lambda i: (0, 0)),
             pl.BlockSpec((1, m), lambda i: (0, 0)),
             pl.BlockSpec((m, 1), lambda i: (0, 0)),
             pl.BlockSpec((m, 1), lambda i: (0, 0)),
             pl.BlockSpec((1, 1), lambda i: (0, 0))]
    if den is not None:
        ins.append(den)
        specs.append(pl.BlockSpec((nb, 128), lambda i: (i, 0)))

    def body(x_ref, w_ref, b_ref, awl_ref, awr_ref, ab_ref, *rest):
        t_ref, tab_ref, mx_ref = rest[-3:]
        d = rest[0][...] if den is not None else None
        i = pl.program_id(0)
        xb = _pre_act(x_ref[...], pre, d)
        t = jnp.dot(xb, w_ref[...], preferred_element_type=F32) + b_ref[...]
        tl = jnp.dot(t, awl_ref[...], preferred_element_type=F32) + ab_ref[...]
        tr = jnp.dot(t, awr_ref[...], preferred_element_type=F32)
        if mo > m:
            t = jnp.concatenate(
                [t, jnp.zeros((t.shape[0], mo - m), F32)], axis=1)
        t_ref[...] = t
        pad = jnp.zeros((xb.shape[0], 63), F32)
        tab_ref[...] = jnp.concatenate([tl, pad, tr, pad], axis=1)
        cur = jnp.concatenate(
            [jnp.max(tl, axis=0, keepdims=True),
             jnp.max(tr, axis=0, keepdims=True),
             jnp.full((1, 14), -BIG, F32)], axis=1)

        @pl.when(i == 0)
        def _():
            mx_ref[...] = cur

        @pl.when(i > 0)
        def _():
            mx_ref[...] = jnp.maximum(mx_ref[...], cur)

    return pl.pallas_call(
        body, grid=(n // nb,), in_specs=specs,
        out_specs=[
            pl.BlockSpec((nb, mo), lambda i: (i, 0)),
            pl.BlockSpec((nb, 128), lambda i: (i, 0)),
            pl.BlockSpec((1, 16), lambda i: (0, 0)),
        ],
        out_shape=[
            jax.ShapeDtypeStruct((n, mo), F32),
            jax.ShapeDtypeStruct((n, 128), F32),
            jax.ShapeDtypeStruct((1, 16), F32),
        ])(*ins)


def _tc_ga_expay(a, b, mx, tg, eb=2048):
    e = a.shape[0]
    m = tg.shape[1]
    nch = m // 128

    def body(a_ref, b_ref, mx_ref, t_ref, ex_ref, *pouts):
        mxv = mx_ref[...]
        s = mxv[:, 0:1] + mxv[:, 1:2]
        s = jnp.maximum(s, 0.2 * s)
        ee = a_ref[..., 0:1] + b_ref[..., 64:65]
        ee = jnp.maximum(ee, 0.2 * ee)
        ex = jnp.exp(ee - s)
        ex_ref[...] = jnp.concatenate(
            [ex, jnp.zeros((ex.shape[0], 127), F32)], axis=1)
        pay = ex * t_ref[...]
        for j in range(nch):
            pouts[j][...] = pay[:, j * 128:(j + 1) * 128]

    return pl.pallas_call(
        body, grid=(e // eb,),
        in_specs=[
            pl.BlockSpec((eb, 128), lambda i: (i, 0)),
            pl.BlockSpec((eb, 128), lambda i: (i, 0)),
            pl.BlockSpec((1, 16), lambda i: (0, 0)),
            pl.BlockSpec((eb, m), lambda i: (i, 0)),
        ],
        out_specs=[pl.BlockSpec((eb, 128), lambda i: (i, 0))] * (1 + nch),
        out_shape=[jax.ShapeDtypeStruct((e, 128), F32)] * (1 + nch))(
            a, b, mx, tg)


def _tc_readout(raw, den, pw, pb, awg, awh, ab, gwi, gwh, gbi, gbh, d):
    n = raw.shape[0]

    def body(r_ref, d_ref, pw_ref, pb_ref, ag_ref, ah_ref, ab_ref,
             wi_ref, wh_ref, bi_ref, bh_ref, out_ref):
        h = r_ref[..., 0:d] / (d_ref[:, 0:1] + EPS)
        hv = jnp.dot(h, pw_ref[...], preferred_element_type=F32) + pb_ref[...]
        lg_h = jnp.dot(h, ah_ref[...], preferred_element_type=F32)
        g = jnp.sum(h, axis=0, keepdims=True)
        for _ in range(3):
            gl = jnp.dot(g, ag_ref[...], preferred_element_type=F32) + ab_ref[...]
            logits = lg_h + gl
            logits = jnp.maximum(logits, 0.01 * logits)
            mxl = jnp.max(logits, axis=0, keepdims=True)
            aa = jnp.exp(logits - mxl)
            aa = aa / jnp.sum(aa, axis=0, keepdims=True)
            context = jnp.sum(aa * hv, axis=0, keepdims=True)
            context = jnp.where(context > 0, context,
                                jnp.exp(jnp.minimum(context, 0.0)) - 1.0)
            gi = jnp.dot(context, wi_ref[...], preferred_element_type=F32) + bi_ref[...]
            gh = jnp.dot(g, wh_ref[...], preferred_element_type=F32) + bh_ref[...]
            rr = jax.nn.sigmoid(gi[:, 0:d] + gh[:, 0:d])
            zz = jax.nn.sigmoid(gi[:, d:2 * d] + gh[:, d:2 * d])
            nn_ = jnp.tanh(gi[:, 2 * d:3 * d] + rr * gh[:, 2 * d:3 * d])
            g = (1.0 - zz) * nn_ + zz * g
        out_ref[...] = g

    return pl.pallas_call(
        body,
        out_shape=jax.ShapeDtypeStruct((1, d), F32))(
            raw, den, pw, pb.reshape(1, d), awg, awh, ab.reshape(1, 1),
            gwi, gwh, gbi.reshape(1, 3 * d), gbh.reshape(1, 3 * d))


def kernel(x, edge_index, edge_attr, params):
    p = params
    n, d_in = x.shape
    e = edge_index.shape[1]
    nhalf = n // 2
    heads, hid = p['gat1_al'].shape
    d_out = p['gat3_al'].shape[1]
    src = edge_index[0]
    dst = edge_index[1]

    dsf = _tc_dstsel(dst.reshape(1, e), n).reshape(4 * e)

    h = _tc_linear(x, p['proj_W'], p['proj_b'], act="relu", out_w=128)
    u = _tc_linear(edge_attr, p['enet_W1'], p['enet_b1'], act="relu", nb=2048)
    kk = u.shape[1]
    w2r = p['enet_W2'].reshape(kk, d_in, d_in)
    w2r = jnp.concatenate(
        [w2r, jnp.zeros((kk, 128 - d_in, d_in), F32)], axis=1)
    b2r = _pad_rows(p['enet_b2'].reshape(d_in, d_in), 128)
    gru_wi = _pad_rows(p['gru_Wi'], 128)
    gru_wh = _pad_rows(p['gru_Wh'], 128)
    hidden = h
    for _ in range(3):
        (hs,) = _sc_gather([h], [src])
        msg = _mp_messages(u, hs, w2r, b2r)
        (agg,) = _sc_scatter([msg], dsf, n)
        h = _tc_gru(agg, hidden, gru_wi, gru_wh, p['gru_bi'], p['gru_bh'],
                    d_in)
        hidden = h

    hcur = h
    for nm, dh, pre, wpad in (("gat1", hid, "none", 128),
                              ("gat2", hid, "elu", 0),
                              ("gat3", d_out, "elu", 0)):
        w = _pad_rows(p[nm + '_W'], wpad) if wpad else p[nm + '_W']
        z, tab, mx = _tc_gat_z(hcur, w, p[nm + '_al'], p[nm + '_ar'],
                               heads, dh, pre=pre)
        a_g, b_g, zg = _sc_gather([tab, tab, z], [src, dst, src])
        ex = _tc_gat_ex(a_g, b_g, mx, heads)
        (den,) = _sc_scatter([ex], dsf, n)
        binv = _tc_binv(den)
        (bg,) = _sc_gather([binv], [dst])
        pays = _tc_gat_pay(zg, ex, bg, heads, dh)
        outs = _sc_scatter(list(pays), dsf, n)
        hcur = outs[0] if len(outs) == 1 else jnp.concatenate(outs, axis=1)

    raw, den_prev = hcur, None
    for nm, wpad in (("ga1", 128), ("ga2", 0), ("ga3", 0)):
        aw = p[nm + '_aw']
        m = aw.shape[0] // 2
        w = _pad_rows(p[nm + '_W'], wpad) if wpad else p[nm + '_W']
        t, tab, mx = _tc_ga_t(raw, w, p[nm + '_b'], aw[:m], aw[m:],
                              p[nm + '_ab'][0], den=den_prev)
        a_g, b_g, tg = _sc_gather([tab, tab, t], [src, dst, src])
        expays = _tc_ga_expay(a_g, b_g, mx, tg)
        souts = _sc_scatter(list(expays), dsf, n)
        den_prev = souts[0]
        raw = (souts[1] if len(souts) == 2
               else jnp.concatenate(souts[1:], axis=1))

    attw = p['ro_attW']
    return _tc_readout(raw, den_prev, p['ro_pW'], p['ro_pb'],
                       attw[:d_out], attw[d_out:], p['ro_attb'][0],
                       p['ro_gWi'], p['ro_gWh'], p['ro_gbi'], p['ro_gbh'],
                       d_out)


# speedup vs baseline: 5.7349x; 1.0104x over previous
import jax
import jax.numpy as jnp
from jax import lax
from jax.experimental import pallas as pl
from jax.experimental.pallas import tpu as pltpu
from jax.experimental.pallas import tpu_sc as plsc

F32 = jnp.float32
NC, NS = 2, 16
NW = NC * NS
BIG = 1e9
EPS = 1e-16


def _sc_mesh():
    return plsc.VectorSubcoreMesh(core_axis_name="c", subcore_axis_name="s")


def _pad_rows(w, rows):
    return jnp.concatenate(
        [w, jnp.zeros((rows - w.shape[0],) + w.shape[1:], F32)], axis=0)


def _sc_gather(tables, idxs):
    e = idxs[0].shape[0]
    ns = len(tables)
    epw = e // NW
    widths = [int(t.shape[1]) for t in tables]
    chs = [64 if w <= 256 else 32 for w in widths]
    out_type = tuple(jax.ShapeDtypeStruct((e, w), F32) for w in widths)
    scratch = []
    for w, ch in zip(widths, chs):
        scratch.append(pltpu.VMEM((2, ch), jnp.int32))
        scratch.append(pltpu.VMEM((2, ch, w), F32))
        scratch.append(pltpu.SemaphoreType.DMA)
        scratch.append(pltpu.SemaphoreType.DMA)

    def body(*refs):
        tabs = refs[:ns]
        idr = refs[ns:2 * ns]
        outs = refs[2 * ns:3 * ns]
        scr = refs[3 * ns:]
        wid = lax.axis_index("s") * NC + lax.axis_index("c")
        base = wid * epw
        for i in range(ns):
            iv, rv = scr[4 * i], scr[4 * i + 1]
            sems = (scr[4 * i + 2], scr[4 * i + 3])
            ch = chs[i]
            nch = epw // ch
            handles = [None, None]
            pltpu.sync_copy(idr[i].at[pl.ds(pl.multiple_of(base, 8), ch)],
                            iv.at[0])
            handles[0] = pltpu.async_copy(tabs[i].at[iv.at[0]], rv.at[0],
                                          sems[0])
            for k in range(nch):
                cur = k & 1
                if k + 1 < nch:
                    nxt = 1 - cur
                    off1 = pl.multiple_of(base + (k + 1) * ch, 8)
                    pltpu.sync_copy(idr[i].at[pl.ds(off1, ch)], iv.at[nxt])
                    handles[nxt] = pltpu.async_copy(
                        tabs[i].at[iv.at[nxt]], rv.at[nxt], sems[nxt])
                handles[cur].wait()
                off = pl.multiple_of(base + k * ch, 8)
                pltpu.sync_copy(rv.at[cur], outs[i].at[pl.ds(off, ch)])

    return pl.kernel(body, out_type=out_type, mesh=_sc_mesh(),
                     scratch_types=tuple(scratch))(*tables, *idxs)


def _sc_scatter(payloads, dst_sel_flat, n):
    e = payloads[0].shape[0]
    ns = len(payloads)
    nq = n // 4
    acc_rows = nq + 128
    stripe = acc_rows // NS
    orows = nq // NS
    ept = e // NS
    ch = 64
    widths = [int(p.shape[1]) for p in payloads]
    out_type = tuple(jax.ShapeDtypeStruct((n, w), F32) for w in widths)
    scratch = [pltpu.VMEM((2, ch), jnp.int32)]
    for w in widths:
        scratch.append(pltpu.VMEM((8, w), F32))
        scratch.append(pltpu.VMEM((2, ch, w), F32))
        scratch.append(pltpu.VMEM_SHARED((acc_rows, w), F32))
        scratch.append(pltpu.SemaphoreType.DMA)
        scratch.append(pltpu.SemaphoreType.DMA)

    def body(dsr, *refs):
        pays = refs[:ns]
        outs = refs[ns:2 * ns]
        scr = refs[2 * ns:]
        iv = scr[0]
        c = lax.axis_index("c")
        s = lax.axis_index("s")
        nch = ept // ch
        for q in range(2):
            for i in range(ns):
                zb, acc = scr[1 + 5 * i], scr[3 + 5 * i]
                if q == 0:
                    for r in range(8):
                        for j in range(widths[i] // 16):
                            zb[r, pl.ds(j * 16, 16)] = jnp.zeros((16,), F32)
                for j in range(stripe // 8):
                    zo = pl.multiple_of(s * stripe + j * 8, 8)
                    pltpu.sync_copy(zb, acc.at[pl.ds(zo, 8)])
            plsc.subcore_barrier()
            handles = [[None, None] for _ in range(ns)]
            for k in range(nch):
                cur = k & 1
                ioff = pl.multiple_of((2 * c + q) * e + s * ept + k * ch, 8)
                eoff = pl.multiple_of(s * ept + k * ch, 8)
                for i in range(ns):
                    if handles[i][cur] is not None:
                        handles[i][cur].wait()
                pltpu.sync_copy(dsr.at[pl.ds(ioff, ch)], iv.at[cur])
                for i in range(ns):
                    pv, acc = scr[2 + 5 * i], scr[3 + 5 * i]
                    sem = scr[4 + 5 * i + cur]
                    pltpu.sync_copy(pays[i].at[pl.ds(eoff, ch)], pv.at[cur])
                    handles[i][cur] = pltpu.async_copy(
                        pv.at[cur], acc.at[iv.at[cur]], sem, add=True)
            for i in range(ns):
                for par in range(2):
                    if handles[i][par] is not None:
                        handles[i][par].wait()
            plsc.subcore_barrier()
            so = pl.multiple_of(s * orows, 8)
            oo = pl.multiple_of((2 * c + q) * nq + s * orows, 8)
            for i in range(ns):
                acc = scr[3 + 5 * i]
                pltpu.sync_copy(acc.at[pl.ds(so, orows)],
                                outs[i].at[pl.ds(oo, orows)])
            plsc.subcore_barrier()

    return pl.kernel(body, out_type=out_type, mesh=_sc_mesh(),
                     scratch_types=tuple(scratch))(dst_sel_flat, *payloads)


def _tc_dstsel(dst2d, n):
    e = dst2d.shape[1]
    nq = n // 4
    eb = 2048

    def body(d_ref, o_ref):
        d = d_ref[...]
        sels = []
        for q in range(4):
            dq = d - q * nq
            sels.append(jnp.where((dq >= 0) & (dq < nq), dq, nq))
        o_ref[...] = jnp.concatenate(sels, axis=0)

    return pl.pallas_call(
        body, grid=(e // eb,),
        in_specs=[pl.BlockSpec((1, eb), lambda i: (0, i))],
        out_specs=pl.BlockSpec((4, eb), lambda i: (0, i)),
        out_shape=jax.ShapeDtypeStruct((4, e), jnp.int32))(dst2d)


def _pre_act(xb, pre, d=None):
    if d is not None:
        xb = xb / (d[:, 0:1] + EPS)
    if pre == "relu":
        xb = jnp.maximum(xb, 0.0)
    elif pre == "elu":
        xb = jnp.where(xb > 0, xb, jnp.exp(jnp.minimum(xb, 0.0)) - 1.0)
    return xb


def _tc_linear(x, w, b, act="none", out_w=None, nb=1024):
    n, kdim = x.shape
    m = w.shape[-1]
    mo = m if out_w is None else out_w

    def body(x_ref, w_ref, b_ref, out_ref):
        y = jnp.dot(x_ref[...], w_ref[...], preferred_element_type=F32)
        y = y + b_ref[...]
        if act == "relu":
            y = jnp.maximum(y, 0.0)
        if mo > m:
            y = jnp.concatenate(
                [y, jnp.zeros((y.shape[0], mo - m), F32)], axis=1)
        out_ref[...] = y

    return pl.pallas_call(
        body, grid=(n // nb,),
        in_specs=[pl.BlockSpec((nb, kdim), lambda i: (i, 0)),
                  pl.BlockSpec((kdim, m), lambda i: (0, 0)),
                  pl.BlockSpec((1, m), lambda i: (0, 0))],
        out_specs=pl.BlockSpec((nb, mo), lambda i: (i, 0)),
        out_shape=jax.ShapeDtypeStruct((n, mo), F32))(x, w, b.reshape(1, m))


def _mp_msg_body(u_ref, hs_ref, w2r_ref, b2r_ref, out_ref):
    u = u_ref[...]
    hs = hs_ref[...]
    d = b2r_ref.shape[1]
    acc = jnp.dot(hs, b2r_ref[...], preferred_element_type=F32)
    for k in range(u.shape[1]):
        acc += u[:, k:k + 1] * jnp.dot(hs, w2r_ref[k],
                                       preferred_element_type=F32)
    acc = jnp.concatenate([acc, jnp.zeros((acc.shape[0], 128 - d), F32)],
                          axis=1)
    out_ref[...] = acc


def _mp_messages(u, hs, w2r, b2r):
    e = hs.shape[0]
    d = b2r.shape[1]
    kk = u.shape[1]
    eb = 2048
    return pl.pallas_call(
        _mp_msg_body,
        grid=(e // eb,),
        in_specs=[
            pl.BlockSpec((eb, kk), lambda i: (i, 0)),
            pl.BlockSpec((eb, 128), lambda i: (i, 0)),
            pl.BlockSpec((kk, 128, d), lambda i: (0, 0, 0)),
            pl.BlockSpec((128, d), lambda i: (0, 0)),
        ],
        out_specs=pl.BlockSpec((eb, 128), lambda i: (i, 0)),
        out_shape=jax.ShapeDtypeStruct((e, 128), F32))(u, hs, w2r, b2r)


def _tc_gru(agg, hidden, wi, wh, bi, bh, d, nb=1024):
    n = hidden.shape[0]

    def body(a_ref, h_ref, wi_ref, wh_ref, bi_ref, bh_ref, out_ref):
        m = jnp.maximum(a_ref[...], 0.0)
        hp = h_ref[...]
        gi = jnp.dot(m, wi_ref[...], preferred_element_type=F32) + bi_ref[...]
        gh = jnp.dot(hp, wh_ref[...], preferred_element_type=F32) + bh_ref[...]
        r = jax.nn.sigmoid(gi[:, 0:d] + gh[:, 0:d])
        z = jax.nn.sigmoid(gi[:, d:2 * d] + gh[:, d:2 * d])
        nn_ = jnp.tanh(gi[:, 2 * d:3 * d] + r * gh[:, 2 * d:3 * d])
        y = (1.0 - z) * nn_ + z * hp[:, 0:d]
        out_ref[...] = jnp.concatenate(
            [y, jnp.zeros((y.shape[0], 128 - d), F32)], axis=1)

    return pl.pallas_call(
        body, grid=(n // nb,),
        in_specs=[
            pl.BlockSpec((nb, 128), lambda i: (i, 0)),
            pl.BlockSpec((nb, 128), lambda i: (i, 0)),
            pl.BlockSpec((128, 3 * d), lambda i: (0, 0)),
            pl.BlockSpec((128, 3 * d), lambda i: (0, 0)),
            pl.BlockSpec((1, 3 * d), lambda i: (0, 0)),
            pl.BlockSpec((1, 3 * d), lambda i: (0, 0)),
        ],
        out_specs=pl.BlockSpec((nb, 128), lambda i: (i, 0)),
        out_shape=jax.ShapeDtypeStruct((n, 128), F32))(
            agg, hidden, wi, wh, bi.reshape(1, 3 * d), bh.reshape(1, 3 * d))


def _tc_gat_z(x, w, al, ar, heads, dh, pre="none", nb=1024):
    n, kdim = x.shape
    m = heads * dh

    def body(x_ref, w_ref, al_ref, ar_ref, z_ref, tab_ref, mx_ref):
        i = pl.program_id(0)
        xb = _pre_act(x_ref[...], pre)
        z = jnp.dot(xb, w_ref[...], preferred_element_type=F32)
        z_ref[...] = z
        els, ers = [], []
        for h in range(heads):
            zh = z[:, h * dh:(h + 1) * dh]
            els.append(jnp.sum(zh * al_ref[h:h + 1, :], axis=1, keepdims=True))
            ers.append(jnp.sum(zh * ar_ref[h:h + 1, :], axis=1, keepdims=True))
        el = jnp.concatenate(els, axis=1)
        er = jnp.concatenate(ers, axis=1)
        pad = jnp.zeros((xb.shape[0], 64 - heads), F32)
        tab_ref[...] = jnp.concatenate([el, pad, er, pad], axis=1)
        cur = jnp.concatenate(
            [jnp.max(el, axis=0, keepdims=True),
             jnp.max(er, axis=0, keepdims=True),
             jnp.full((1, 16 - 2 * heads), -BIG, F32)], axis=1)

        @pl.when(i == 0)
        def _():
            mx_ref[...] = cur

        @pl.when(i > 0)
        def _():
            mx_ref[...] = jnp.maximum(mx_ref[...], cur)

    return pl.pallas_call(
        body, grid=(n // nb,),
        in_specs=[
            pl.BlockSpec((nb, kdim), lambda i: (i, 0)),
            pl.BlockSpec((kdim, m), lambda i: (0, 0)),
            pl.BlockSpec((heads, dh), lambda i: (0, 0)),
            pl.BlockSpec((heads, dh), lambda i: (0, 0)),
        ],
        out_specs=[
            pl.BlockSpec((nb, m), lambda i: (i, 0)),
            pl.BlockSpec((nb, 128), lambda i: (i, 0)),
            pl.BlockSpec((1, 16), lambda i: (0, 0)),
        ],
        out_shape=[
            jax.ShapeDtypeStruct((n, m), F32),
            jax.ShapeDtypeStruct((n, 128), F32),
            jax.ShapeDtypeStruct((1, 16), F32),
        ])(x, w, al, ar)


def _tc_gat_ex(a, b, mx, heads, eb=2048):
    e = a.shape[0]

    def body(a_ref, b_ref, mx_ref, out_ref):
        m = mx_ref[...]
        s = m[:, 0:heads] + m[:, heads:2 * heads]
        s = jnp.maximum(s, 0.2 * s)
        ee = a_ref[..., 0:heads] + b_ref[..., 64:64 + heads]
        ee = jnp.maximum(ee, 0.2 * ee)
        ex = jnp.exp(ee - s)
        out_ref[...] = jnp.concatenate(
            [ex, jnp.zeros((ex.shape[0], 128 - heads), F32)], axis=1)

    return pl.pallas_call(
        body, grid=(e // eb,),
        in_specs=[
            pl.BlockSpec((eb, 128), lambda i: (i, 0)),
            pl.BlockSpec((eb, 128), lambda i: (i, 0)),
            pl.BlockSpec((1, 16), lambda i: (0, 0)),
        ],
        out_specs=pl.BlockSpec((eb, 128), lambda i: (i, 0)),
        out_shape=jax.ShapeDtypeStruct((e, 128), F32))(a, b, mx)


def _tc_binv(den, nb=1024):
    n = den.shape[0]

    def body(d_ref, o_ref):
        o_ref[...] = 1.0 / (d_ref[...] + EPS)

    return pl.pallas_call(
        body, grid=(n // nb,),
        in_specs=[pl.BlockSpec((nb, 128), lambda i: (i, 0))],
        out_specs=pl.BlockSpec((nb, 128), lambda i: (i, 0)),
        out_shape=jax.ShapeDtypeStruct((n, 128), F32))(den)


def _tc_gat_pay(zg, ex, bg, heads, dh, eb=2048):
    e, m = zg.shape
    mo = max(dh, 128)
    nch = mo // 128

    def body(z_ref, e_ref, b_ref, *outs):
        w = e_ref[..., 0:heads] * b_ref[..., 0:heads] * (1.0 / heads)
        z = z_ref[...]
        acc = w[:, 0:1] * z[:, 0:dh]
        for h in range(1, heads):
            acc += w[:, h:h + 1] * z[:, h * dh:(h + 1) * dh]
        if mo > dh:
            acc = jnp.concatenate(
                [acc, jnp.zeros((acc.shape[0], mo - dh), F32)], axis=1)
        for j in range(nch):
            outs[j][...] = acc[:, j * 128:(j + 1) * 128]

    return pl.pallas_call(
        body, grid=(e // eb,),
        in_specs=[
            pl.BlockSpec((eb, m), lambda i: (i, 0)),
            pl.BlockSpec((eb, 128), lambda i: (i, 0)),
            pl.BlockSpec((eb, 128), lambda i: (i, 0)),
        ],
        out_specs=[pl.BlockSpec((eb, 128), lambda i: (i, 0))] * nch,
        out_shape=[jax.ShapeDtypeStruct((e, 128), F32)] * nch)(zg, ex, bg)


def _tc_ga_t(x, w, b, awl, awr, ab, pre="none", den=None, nb=1024):
    n, kdim = x.shape
    m = w.shape[1]
    mo = max(m, 128)
    ins = [x, w, b.reshape(1, m), awl, awr, ab.reshape(1, 1)]
    specs = [pl.BlockSpec((nb, kdim), lambda i: (i, 0)),
             pl.BlockSpec((kdim, m), lambda i: (0, 0)),
             pl.BlockSpec((1, m), lambda i: (0, 0)),
             pl.BlockSpec((m, 1), lambda i: (0, 0)),
             pl.BlockSpec((m, 1), lambda i: (0, 0)),
             pl.BlockSpec((1, 1), lambda i: (0, 0))]
    if den is not None:
        ins.append(den)
        specs.append(pl.BlockSpec((nb, 128), lambda i: (i, 0)))

    def body(x_ref, w_ref, b_ref, awl_ref, awr_ref, ab_ref, *rest):
        t_ref, tab_ref, mx_ref = rest[-3:]
        d = rest[0][...] if den is not None else None
        i = pl.program_id(0)
        xb = _pre_act(x_ref[...], pre, d)
        t = jnp.dot(xb, w_ref[...], preferred_element_type=F32) + b_ref[...]
        tl = jnp.dot(t, awl_ref[...], preferred_element_type=F32) + ab_ref[...]
        tr = jnp.dot(t, awr_ref[...], preferred_element_type=F32)
        if mo > m:
            t = jnp.concatenate(
                [t, jnp.zeros((t.shape[0], mo - m), F32)], axis=1)
        t_ref[...] = t
        pad = jnp.zeros((xb.shape[0], 63), F32)
        tab_ref[...] = jnp.concatenate([tl, pad, tr, pad], axis=1)
        cur = jnp.concatenate(
            [jnp.max(tl, axis=0, keepdims=True),
             jnp.max(tr, axis=0, keepdims=True),
             jnp.full((1, 14), -BIG, F32)], axis=1)

        @pl.when(i == 0)
        def _():
            mx_ref[...] = cur

        @pl.when(i > 0)
        def _():
            mx_ref[...] = jnp.maximum(mx_ref[...], cur)

    return pl.pallas_call(
        body, grid=(n // nb,), in_specs=specs,
        out_specs=[
            pl.BlockSpec((nb, mo), lambda i: (i, 0)),
            pl.BlockSpec((nb, 128), lambda i: (i, 0)),
            pl.BlockSpec((1, 16), lambda i: (0, 0)),
        ],
        out_shape=[
            jax.ShapeDtypeStruct((n, mo), F32),
            jax.ShapeDtypeStruct((n, 128), F32),
            jax.ShapeDtypeStruct((1, 16), F32),
        ])(*ins)


def _tc_ga_expay(a, b, mx, tg, eb=2048):
    e = a.shape[0]
    m = tg.shape[1]
    nch = m // 128

    def body(a_ref, b_ref, mx_ref, t_ref, ex_ref, *pouts):
        mxv = mx_ref[...]
        s = mxv[:, 0:1] + mxv[:, 1:2]
        s = jnp.maximum(s, 0.2 * s)
        ee = a_ref[..., 0:1] + b_ref[..., 64:65]
        ee = jnp.maximum(ee, 0.2 * ee)
        ex = jnp.exp(ee - s)
        ex_ref[...] = jnp.concatenate(
            [ex, jnp.zeros((ex.shape[0], 127), F32)], axis=1)
        pay = ex * t_ref[...]
        for j in range(nch):
            pouts[j][...] = pay[:, j * 128:(j + 1) * 128]

    return pl.pallas_call(
        body, grid=(e // eb,),
        in_specs=[
            pl.BlockSpec((eb, 128), lambda i: (i, 0)),
            pl.BlockSpec((eb, 128), lambda i: (i, 0)),
            pl.BlockSpec((1, 16), lambda i: (0, 0)),
            pl.BlockSpec((eb, m), lambda i: (i, 0)),
        ],
        out_specs=[pl.BlockSpec((eb, 128), lambda i: (i, 0))] * (1 + nch),
        out_shape=[jax.ShapeDtypeStruct((e, 128), F32)] * (1 + nch))(
            a, b, mx, tg)


def _tc_readout(raw, den, pw, pb, awg, awh, ab, gwi, gwh, gbi, gbh, d):
    n = raw.shape[0]

    def body(r_ref, d_ref, pw_ref, pb_ref, ag_ref, ah_ref, ab_ref,
             wi_ref, wh_ref, bi_ref, bh_ref, out_ref):
        h = r_ref[..., 0:d] / (d_ref[:, 0:1] + EPS)
        hv = jnp.dot(h, pw_ref[...], preferred_element_type=F32) + pb_ref[...]
        lg_h = jnp.dot(h, ah_ref[...], preferred_element_type=F32)
        g = jnp.sum(h, axis=0, keepdims=True)
        for _ in range(3):
            gl = jnp.dot(g, ag_ref[...], preferred_element_type=F32) + ab_ref[...]
            logits = lg_h + gl
            logits = jnp.maximum(logits, 0.01 * logits)
            mxl = jnp.max(logits, axis=0, keepdims=True)
            aa = jnp.exp(logits - mxl)
            aa = aa / jnp.sum(aa, axis=0, keepdims=True)
            context = jnp.sum(aa * hv, axis=0, keepdims=True)
            context = jnp.where(context > 0, context,
                                jnp.exp(jnp.minimum(context, 0.0)) - 1.0)
            gi = jnp.dot(context, wi_ref[...], preferred_element_type=F32) + bi_ref[...]
            gh = jnp.dot(g, wh_ref[...], preferred_element_type=F32) + bh_ref[...]
            rr = jax.nn.sigmoid(gi[:, 0:d] + gh[:, 0:d])
            zz = jax.nn.sigmoid(gi[:, d:2 * d] + gh[:, d:2 * d])
            nn_ = jnp.tanh(gi[:, 2 * d:3 * d] + rr * gh[:, 2 * d:3 * d])
            g = (1.0 - zz) * nn_ + zz * g
        out_ref[...] = g

    return pl.pallas_call(
        body,
        out_shape=jax.ShapeDtypeStruct((1, d), F32))(
            raw, den, pw, pb.reshape(1, d), awg, awh, ab.reshape(1, 1),
            gwi, gwh, gbi.reshape(1, 3 * d), gbh.reshape(1, 3 * d))


def kernel(x, edge_index, edge_attr, params):
    p = params
    n, d_in = x.shape
    e = edge_index.shape[1]
    nhalf = n // 2
    heads, hid = p['gat1_al'].shape
    d_out = p['gat3_al'].shape[1]
    src = edge_index[0]
    dst = edge_index[1]

    dsf = _tc_dstsel(dst.reshape(1, e), n).reshape(4 * e)

    h = _tc_linear(x, p['proj_W'], p['proj_b'], act="relu", out_w=128)
    u = _tc_linear(edge_attr, p['enet_W1'], p['enet_b1'], act="relu", nb=2048)
    kk = u.shape[1]
    w2r = p['enet_W2'].reshape(kk, d_in, d_in)
    w2r = jnp.concatenate(
        [w2r, jnp.zeros((kk, 128 - d_in, d_in), F32)], axis=1)
    b2r = _pad_rows(p['enet_b2'].reshape(d_in, d_in), 128)
    gru_wi = _pad_rows(p['gru_Wi'], 128)
    gru_wh = _pad_rows(p['gru_Wh'], 128)
    hidden = h
    for _ in range(3):
        (hs,) = _sc_gather([h], [src])
        msg = _mp_messages(u, hs, w2r, b2r)
        (agg,) = _sc_scatter([msg], dsf, n)
        h = _tc_gru(agg, hidden, gru_wi, gru_wh, p['gru_bi'], p['gru_bh'],
                    d_in)
        hidden = h

    hcur = h
    for nm, dh, pre, wpad in (("gat1", hid, "none", 128),
                              ("gat2", hid, "elu", 0),
                              ("gat3", d_out, "elu", 0)):
        w = _pad_rows(p[nm + '_W'], wpad) if wpad else p[nm + '_W']
        z, tab, mx = _tc_gat_z(hcur, w, p[nm + '_al'], p[nm + '_ar'],
                               heads, dh, pre=pre)
        a_g, b_g, zg = _sc_gather([tab, tab, z], [src, dst, src])
        ex = _tc_gat_ex(a_g, b_g, mx, heads)
        (den,) = _sc_scatter([ex], dsf, n)
        binv = _tc_binv(den)
        (bg,) = _sc_gather([binv], [dst])
        pays = _tc_gat_pay(zg, ex, bg, heads, dh)
        outs = _sc_scatter(list(pays), dsf, n)
        hcur = outs[0] if len(outs) == 1 else jnp.concatenate(outs, axis=1)

    raw, den_prev = hcur, None
    for nm, wpad in (("ga1", 128), ("ga2", 0), ("ga3", 0)):
        aw = p[nm + '_aw']
        m = aw.shape[0] // 2
        w = _pad_rows(p[nm + '_W'], wpad) if wpad else p[nm + '_W']
        t, tab, mx = _tc_ga_t(raw, w, p[nm + '_b'], aw[:m], aw[m:],
                              p[nm + '_ab'][0], den=den_prev)
        a_g, b_g, tg = _sc_gather([tab, tab, t], [src, dst, src])
        expays = _tc_ga_expay(a_g, b_g, mx, tg)
        souts = _sc_scatter(list(expays), dsf, n)
        den_prev = souts[0]
        raw = (souts[1] if len(souts) == 2
               else jnp.concatenate(souts[1:], axis=1))

    attw = p['ro_attW']
    return _tc_readout(raw, den_prev, p['ro_pW'], p['ro_pb'],
                       attw[:d_out], attw[d_out:], p['ro_attb'][0],
                       p['ro_gWi'], p['ro_gWh'], p['ro_gbi'], p['ro_gbh'],
                       d_out)


# speedup vs baseline: 6.7238x; 1.1724x over previous
import jax
import jax.numpy as jnp
from jax import lax
from jax.experimental import pallas as pl
from jax.experimental.pallas import tpu as pltpu
from jax.experimental.pallas import tpu_sc as plsc

F32 = jnp.float32
NC, NS = 2, 16
NW = NC * NS
BIG = 1e9
EPS = 1e-16


def _sc_mesh():
    return plsc.VectorSubcoreMesh(core_axis_name="c", subcore_axis_name="s")


def _pad_rows(w, rows):
    return jnp.concatenate(
        [w, jnp.zeros((rows - w.shape[0],) + w.shape[1:], F32)], axis=0)


def _sc_gather(tables, idxs):
    e = idxs[0].shape[0]
    ns = len(tables)
    epw = e // NW
    widths = [int(t.shape[1]) for t in tables]
    chs = [64 if w <= 256 else 32 for w in widths]
    out_type = tuple(jax.ShapeDtypeStruct((e, w), F32) for w in widths)
    scratch = []
    for w, ch in zip(widths, chs):
        scratch.append(pltpu.VMEM((2, ch), jnp.int32))
        scratch.append(pltpu.VMEM((2, ch, w), F32))
        scratch.append(pltpu.SemaphoreType.DMA)
        scratch.append(pltpu.SemaphoreType.DMA)

    def body(*refs):
        tabs = refs[:ns]
        idr = refs[ns:2 * ns]
        outs = refs[2 * ns:3 * ns]
        scr = refs[3 * ns:]
        wid = lax.axis_index("s") * NC + lax.axis_index("c")
        base = wid * epw
        for i in range(ns):
            iv, rv = scr[4 * i], scr[4 * i + 1]
            sems = (scr[4 * i + 2], scr[4 * i + 3])
            ch = chs[i]
            nch = epw // ch
            handles = [None, None]
            pltpu.sync_copy(idr[i].at[pl.ds(pl.multiple_of(base, 8), ch)],
                            iv.at[0])
            handles[0] = pltpu.async_copy(tabs[i].at[iv.at[0]], rv.at[0],
                                          sems[0])
            for k in range(nch):
                cur = k & 1
                if k + 1 < nch:
                    nxt = 1 - cur
                    off1 = pl.multiple_of(base + (k + 1) * ch, 8)
                    pltpu.sync_copy(idr[i].at[pl.ds(off1, ch)], iv.at[nxt])
                    handles[nxt] = pltpu.async_copy(
                        tabs[i].at[iv.at[nxt]], rv.at[nxt], sems[nxt])
                handles[cur].wait()
                off = pl.multiple_of(base + k * ch, 8)
                pltpu.sync_copy(rv.at[cur], outs[i].at[pl.ds(off, ch)])

    return pl.kernel(body, out_type=out_type, mesh=_sc_mesh(),
                     scratch_types=tuple(scratch))(*tables, *idxs)


def _sc_scatter(payloads, dst_sel_flat, n, parts=2):
    e = payloads[0].shape[0]
    ns = len(payloads)
    phases = parts // NC
    nq = n // parts
    acc_rows = nq + 128
    stripe = acc_rows // NS
    orows = nq // NS
    ept = e // NS
    ch = 64
    widths = [int(p.shape[1]) for p in payloads]
    out_type = tuple(jax.ShapeDtypeStruct((n, w), F32) for w in widths)
    scratch = [pltpu.VMEM((2, ch), jnp.int32)]
    for w in widths:
        scratch.append(pltpu.VMEM((8, w), F32))
        scratch.append(pltpu.VMEM((2, ch, w), F32))
        scratch.append(pltpu.VMEM_SHARED((acc_rows, w), F32))
        scratch.append(pltpu.SemaphoreType.DMA)
        scratch.append(pltpu.SemaphoreType.DMA)

    def body(dsr, *refs):
        pays = refs[:ns]
        outs = refs[ns:2 * ns]
        scr = refs[2 * ns:]
        iv = scr[0]
        c = lax.axis_index("c")
        s = lax.axis_index("s")
        nch = ept // ch
        for q in range(phases):
            for i in range(ns):
                zb, acc = scr[1 + 5 * i], scr[3 + 5 * i]
                if q == 0:
                    for r in range(8):
                        for j in range(widths[i] // 16):
                            zb[r, pl.ds(j * 16, 16)] = jnp.zeros((16,), F32)
                for j in range(stripe // 8):
                    zo = pl.multiple_of(s * stripe + j * 8, 8)
                    pltpu.sync_copy(zb, acc.at[pl.ds(zo, 8)])
            plsc.subcore_barrier()
            handles = [[None, None] for _ in range(ns)]
            for k in range(nch):
                cur = k & 1
                ioff = pl.multiple_of(
                    (phases * c + q) * e + s * ept + k * ch, 8)
                eoff = pl.multiple_of(s * ept + k * ch, 8)
                for i in range(ns):
                    if handles[i][cur] is not None:
                        handles[i][cur].wait()
                pltpu.sync_copy(dsr.at[pl.ds(ioff, ch)], iv.at[cur])
                for i in range(ns):
                    pv, acc = scr[2 + 5 * i], scr[3 + 5 * i]
                    sem = scr[4 + 5 * i + cur]
                    pltpu.sync_copy(pays[i].at[pl.ds(eoff, ch)], pv.at[cur])
                    handles[i][cur] = pltpu.async_copy(
                        pv.at[cur], acc.at[iv.at[cur]], sem, add=True)
            for i in range(ns):
                for par in range(2):
                    if handles[i][par] is not None:
                        handles[i][par].wait()
            plsc.subcore_barrier()
            so = pl.multiple_of(s * orows, 8)
            oo = pl.multiple_of((phases * c + q) * nq + s * orows, 8)
            for i in range(ns):
                acc = scr[3 + 5 * i]
                pltpu.sync_copy(acc.at[pl.ds(so, orows)],
                                outs[i].at[pl.ds(oo, orows)])
            plsc.subcore_barrier()

    return pl.kernel(body, out_type=out_type, mesh=_sc_mesh(),
                     scratch_types=tuple(scratch))(dst_sel_flat, *payloads)


def _tc_dstsel(dst2d, n):
    e = dst2d.shape[1]
    eb = 2048

    def body(d_ref, oh_ref, oq_ref):
        d = d_ref[...]
        for parts, o_ref in ((2, oh_ref), (4, oq_ref)):
            r = n // parts
            sels = []
            for q in range(parts):
                dq = d - q * r
                sels.append(jnp.where((dq >= 0) & (dq < r), dq, r))
            o_ref[...] = jnp.concatenate(sels, axis=0)

    return pl.pallas_call(
        body, grid=(e // eb,),
        in_specs=[pl.BlockSpec((1, eb), lambda i: (0, i))],
        out_specs=[pl.BlockSpec((2, eb), lambda i: (0, i)),
                   pl.BlockSpec((4, eb), lambda i: (0, i))],
        out_shape=[jax.ShapeDtypeStruct((2, e), jnp.int32),
                   jax.ShapeDtypeStruct((4, e), jnp.int32)])(dst2d)


def _pre_act(xb, pre, d=None):
    if d is not None:
        xb = xb / (d[:, 0:1] + EPS)
    if pre == "relu":
        xb = jnp.maximum(xb, 0.0)
    elif pre == "elu":
        xb = jnp.where(xb > 0, xb, jnp.exp(jnp.minimum(xb, 0.0)) - 1.0)
    return xb


def _tc_linear(x, w, b, act="none", out_w=None, nb=1024):
    n, kdim = x.shape
    m = w.shape[-1]
    mo = m if out_w is None else out_w

    def body(x_ref, w_ref, b_ref, out_ref):
        y = jnp.dot(x_ref[...], w_ref[...], preferred_element_type=F32)
        y = y + b_ref[...]
        if act == "relu":
            y = jnp.maximum(y, 0.0)
        if mo > m:
            y = jnp.concatenate(
                [y, jnp.zeros((y.shape[0], mo - m), F32)], axis=1)
        out_ref[...] = y

    return pl.pallas_call(
        body, grid=(n // nb,),
        in_specs=[pl.BlockSpec((nb, kdim), lambda i: (i, 0)),
                  pl.BlockSpec((kdim, m), lambda i: (0, 0)),
                  pl.BlockSpec((1, m), lambda i: (0, 0))],
        out_specs=pl.BlockSpec((nb, mo), lambda i: (i, 0)),
        out_shape=jax.ShapeDtypeStruct((n, mo), F32))(x, w, b.reshape(1, m))


def _mp_msg_body(u_ref, hs_ref, w2r_ref, b2r_ref, out_ref):
    u = u_ref[...]
    hs = hs_ref[...]
    d = b2r_ref.shape[1]
    acc = jnp.dot(hs, b2r_ref[...], preferred_element_type=F32)
    for k in range(u.shape[1]):
        acc += u[:, k:k + 1] * jnp.dot(hs, w2r_ref[k],
                                       preferred_element_type=F32)
    acc = jnp.concatenate([acc, jnp.zeros((acc.shape[0], 128 - d), F32)],
                          axis=1)
    out_ref[...] = acc


def _mp_messages(u, hs, w2r, b2r):
    e = hs.shape[0]
    d = b2r.shape[1]
    kk = u.shape[1]
    eb = 2048
    return pl.pallas_call(
        _mp_msg_body,
        grid=(e // eb,),
        in_specs=[
            pl.BlockSpec((eb, kk), lambda i: (i, 0)),
            pl.BlockSpec((eb, 128), lambda i: (i, 0)),
            pl.BlockSpec((kk, 128, d), lambda i: (0, 0, 0)),
            pl.BlockSpec((128, d), lambda i: (0, 0)),
        ],
        out_specs=pl.BlockSpec((eb, 128), lambda i: (i, 0)),
        out_shape=jax.ShapeDtypeStruct((e, 128), F32))(u, hs, w2r, b2r)


def _tc_gru(agg, hidden, wi, wh, bi, bh, d, nb=1024):
    n = hidden.shape[0]

    def body(a_ref, h_ref, wi_ref, wh_ref, bi_ref, bh_ref, out_ref):
        m = jnp.maximum(a_ref[...], 0.0)
        hp = h_ref[...]
        gi = jnp.dot(m, wi_ref[...], preferred_element_type=F32) + bi_ref[...]
        gh = jnp.dot(hp, wh_ref[...], preferred_element_type=F32) + bh_ref[...]
        r = jax.nn.sigmoid(gi[:, 0:d] + gh[:, 0:d])
        z = jax.nn.sigmoid(gi[:, d:2 * d] + gh[:, d:2 * d])
        nn_ = jnp.tanh(gi[:, 2 * d:3 * d] + r * gh[:, 2 * d:3 * d])
        y = (1.0 - z) * nn_ + z * hp[:, 0:d]
        out_ref[...] = jnp.concatenate(
            [y, jnp.zeros((y.shape[0], 128 - d), F32)], axis=1)

    return pl.pallas_call(
        body, grid=(n // nb,),
        in_specs=[
            pl.BlockSpec((nb, 128), lambda i: (i, 0)),
            pl.BlockSpec((nb, 128), lambda i: (i, 0)),
            pl.BlockSpec((128, 3 * d), lambda i: (0, 0)),
            pl.BlockSpec((128, 3 * d), lambda i: (0, 0)),
            pl.BlockSpec((1, 3 * d), lambda i: (0, 0)),
            pl.BlockSpec((1, 3 * d), lambda i: (0, 0)),
        ],
        out_specs=pl.BlockSpec((nb, 128), lambda i: (i, 0)),
        out_shape=jax.ShapeDtypeStruct((n, 128), F32))(
            agg, hidden, wi, wh, bi.reshape(1, 3 * d), bh.reshape(1, 3 * d))


def _tc_gat_z(x, w, al, ar, heads, dh, pre="none", nb=1024):
    n, kdim = x.shape
    m = heads * dh

    def body(x_ref, w_ref, al_ref, ar_ref, z_ref, tab_ref, mx_ref):
        i = pl.program_id(0)
        xb = _pre_act(x_ref[...], pre)
        z = jnp.dot(xb, w_ref[...], preferred_element_type=F32)
        z_ref[...] = z
        els, ers = [], []
        for h in range(heads):
            zh = z[:, h * dh:(h + 1) * dh]
            els.append(jnp.sum(zh * al_ref[h:h + 1, :], axis=1, keepdims=True))
            ers.append(jnp.sum(zh * ar_ref[h:h + 1, :], axis=1, keepdims=True))
        el = jnp.concatenate(els, axis=1)
        er = jnp.concatenate(ers, axis=1)
        pad = jnp.zeros((xb.shape[0], 64 - heads), F32)
        tab_ref[...] = jnp.concatenate([el, pad, er, pad], axis=1)
        cur = jnp.concatenate(
            [jnp.max(el, axis=0, keepdims=True),
             jnp.max(er, axis=0, keepdims=True),
             jnp.full((1, 16 - 2 * heads), -BIG, F32)], axis=1)

        @pl.when(i == 0)
        def _():
            mx_ref[...] = cur

        @pl.when(i > 0)
        def _():
            mx_ref[...] = jnp.maximum(mx_ref[...], cur)

    return pl.pallas_call(
        body, grid=(n // nb,),
        in_specs=[
            pl.BlockSpec((nb, kdim), lambda i: (i, 0)),
            pl.BlockSpec((kdim, m), lambda i: (0, 0)),
            pl.BlockSpec((heads, dh), lambda i: (0, 0)),
            pl.BlockSpec((heads, dh), lambda i: (0, 0)),
        ],
        out_specs=[
            pl.BlockSpec((nb, m), lambda i: (i, 0)),
            pl.BlockSpec((nb, 128), lambda i: (i, 0)),
            pl.BlockSpec((1, 16), lambda i: (0, 0)),
        ],
        out_shape=[
            jax.ShapeDtypeStruct((n, m), F32),
            jax.ShapeDtypeStruct((n, 128), F32),
            jax.ShapeDtypeStruct((1, 16), F32),
        ])(x, w, al, ar)


def _tc_gat_ex(a, b, mx, heads, eb=2048):
    e = a.shape[0]

    def body(a_ref, b_ref, mx_ref, out_ref):
        m = mx_ref[...]
        s = m[:, 0:heads] + m[:, heads:2 * heads]
        s = jnp.maximum(s, 0.2 * s)
        ee = a_ref[..., 0:heads] + b_ref[..., 64:64 + heads]
        ee = jnp.maximum(ee, 0.2 * ee)
        ex = jnp.exp(ee - s)
        out_ref[...] = jnp.concatenate(
            [ex, jnp.zeros((ex.shape[0], 128 - heads), F32)], axis=1)

    return pl.pallas_call(
        body, grid=(e // eb,),
        in_specs=[
            pl.BlockSpec((eb, 128), lambda i: (i, 0)),
            pl.BlockSpec((eb, 128), lambda i: (i, 0)),
            pl.BlockSpec((1, 16), lambda i: (0, 0)),
        ],
        out_specs=pl.BlockSpec((eb, 128), lambda i: (i, 0)),
        out_shape=jax.ShapeDtypeStruct((e, 128), F32))(a, b, mx)


def _tc_binv(den, nb=1024):
    n = den.shape[0]

    def body(d_ref, o_ref):
        o_ref[...] = 1.0 / (d_ref[...] + EPS)

    return pl.pallas_call(
        body, grid=(n // nb,),
        in_specs=[pl.BlockSpec((nb, 128), lambda i: (i, 0))],
        out_specs=pl.BlockSpec((nb, 128), lambda i: (i, 0)),
        out_shape=jax.ShapeDtypeStruct((n, 128), F32))(den)


def _tc_gat_pay(zg, ex, bg, heads, dh, eb=2048):
    e, m = zg.shape
    mo = max(dh, 128)
    nch = mo // 128

    def body(z_ref, e_ref, b_ref, *outs):
        w = (e_ref[..., 0:heads] / (b_ref[..., 0:heads] + EPS)
             * (1.0 / heads))
        z = z_ref[...]
        acc = w[:, 0:1] * z[:, 0:dh]
        for h in range(1, heads):
            acc += w[:, h:h + 1] * z[:, h * dh:(h + 1) * dh]
        if mo > dh:
            acc = jnp.concatenate(
                [acc, jnp.zeros((acc.shape[0], mo - dh), F32)], axis=1)
        for j in range(nch):
            outs[j][...] = acc[:, j * 128:(j + 1) * 128]

    return pl.pallas_call(
        body, grid=(e // eb,),
        in_specs=[
            pl.BlockSpec((eb, m), lambda i: (i, 0)),
            pl.BlockSpec((eb, 128), lambda i: (i, 0)),
            pl.BlockSpec((eb, 128), lambda i: (i, 0)),
        ],
        out_specs=[pl.BlockSpec((eb, 128), lambda i: (i, 0))] * nch,
        out_shape=[jax.ShapeDtypeStruct((e, 128), F32)] * nch)(zg, ex, bg)


def _tc_ga_t(x, w, b, awl, awr, ab, pre="none", den=None, nb=1024):
    n, kdim = x.shape
    m = w.shape[1]
    mo = max(m, 128)
    ins = [x, w, b.reshape(1, m), awl, awr, ab.reshape(1, 1)]
    specs = [pl.BlockSpec((nb, kdim), lambda i: (i, 0)),
             pl.BlockSpec((kdim, m), lambda i: (0, 0)),
             pl.BlockSpec((1, m), lambda i: (0, 0)),
             pl.BlockSpec((m, 1), lambda i: (0, 0)),
             pl.BlockSpec((m, 1), lambda i: (0, 0)),
             pl.BlockSpec((1, 1), lambda i: (0, 0))]
    if den is not None:
        ins.append(den)
        specs.append(pl.BlockSpec((nb, 128), lambda i: (i, 0)))

    def body(x_ref, w_ref, b_ref, awl_ref, awr_ref, ab_ref, *rest):
        t_ref, tab_ref, mx_ref = rest[-3:]
        d = rest[0][...] if den is not None else None
        i = pl.program_id(0)
        xb = _pre_act(x_ref[...], pre, d)
        t = jnp.dot(xb, w_ref[...], preferred_element_type=F32) + b_ref[...]
        tl = jnp.dot(t, awl_ref[...], preferred_element_type=F32) + ab_ref[...]
        tr = jnp.dot(t, awr_ref[...], preferred_element_type=F32)
        if mo > m:
            t = jnp.concatenate(
                [t, jnp.zeros((t.shape[0], mo - m), F32)], axis=1)
        t_ref[...] = t
        pad = jnp.zeros((xb.shape[0], 63), F32)
        tab_ref[...] = jnp.concatenate([tl, pad, tr, pad], axis=1)
        cur = jnp.concatenate(
            [jnp.max(tl, axis=0, keepdims=True),
             jnp.max(tr, axis=0, keepdims=True),
             jnp.full((1, 14), -BIG, F32)], axis=1)

        @pl.when(i == 0)
        def _():
            mx_ref[...] = cur

        @pl.when(i > 0)
        def _():
            mx_ref[...] = jnp.maximum(mx_ref[...], cur)

    return pl.pallas_call(
        body, grid=(n // nb,), in_specs=specs,
        out_specs=[
            pl.BlockSpec((nb, mo), lambda i: (i, 0)),
            pl.BlockSpec((nb, 128), lambda i: (i, 0)),
            pl.BlockSpec((1, 16), lambda i: (0, 0)),
        ],
        out_shape=[
            jax.ShapeDtypeStruct((n, mo), F32),
            jax.ShapeDtypeStruct((n, 128), F32),
            jax.ShapeDtypeStruct((1, 16), F32),
        ])(*ins)


def _tc_ga_expay(a, b, mx, tg, eb=2048):
    e = a.shape[0]
    m = tg.shape[1]
    nch = m // 128

    def body(a_ref, b_ref, mx_ref, t_ref, ex_ref, *pouts):
        mxv = mx_ref[...]
        s = mxv[:, 0:1] + mxv[:, 1:2]
        s = jnp.maximum(s, 0.2 * s)
        ee = a_ref[..., 0:1] + b_ref[..., 64:65]
        ee = jnp.maximum(ee, 0.2 * ee)
        ex = jnp.exp(ee - s)
        ex_ref[...] = jnp.concatenate(
            [ex, jnp.zeros((ex.shape[0], 127), F32)], axis=1)
        pay = ex * t_ref[...]
        for j in range(nch):
            pouts[j][...] = pay[:, j * 128:(j + 1) * 128]

    return pl.pallas_call(
        body, grid=(e // eb,),
        in_specs=[
            pl.BlockSpec((eb, 128), lambda i: (i, 0)),
            pl.BlockSpec((eb, 128), lambda i: (i, 0)),
            pl.BlockSpec((1, 16), lambda i: (0, 0)),
            pl.BlockSpec((eb, m), lambda i: (i, 0)),
        ],
        out_specs=[pl.BlockSpec((eb, 128), lambda i: (i, 0))] * (1 + nch),
        out_shape=[jax.ShapeDtypeStruct((e, 128), F32)] * (1 + nch))(
            a, b, mx, tg)


def _tc_readout(raw, den, pw, pb, awg, awh, ab, gwi, gwh, gbi, gbh, d):
    n = raw.shape[0]

    def body(r_ref, d_ref, pw_ref, pb_ref, ag_ref, ah_ref, ab_ref,
             wi_ref, wh_ref, bi_ref, bh_ref, out_ref):
        h = r_ref[..., 0:d] / (d_ref[:, 0:1] + EPS)
        hv = jnp.dot(h, pw_ref[...], preferred_element_type=F32) + pb_ref[...]
        lg_h = jnp.dot(h, ah_ref[...], preferred_element_type=F32)
        g = jnp.sum(h, axis=0, keepdims=True)
        for _ in range(3):
            gl = jnp.dot(g, ag_ref[...], preferred_element_type=F32) + ab_ref[...]
            logits = lg_h + gl
            logits = jnp.maximum(logits, 0.01 * logits)
            mxl = jnp.max(logits, axis=0, keepdims=True)
            aa = jnp.exp(logits - mxl)
            aa = aa / jnp.sum(aa, axis=0, keepdims=True)
            context = jnp.sum(aa * hv, axis=0, keepdims=True)
            context = jnp.where(context > 0, context,
                                jnp.exp(jnp.minimum(context, 0.0)) - 1.0)
            gi = jnp.dot(context, wi_ref[...], preferred_element_type=F32) + bi_ref[...]
            gh = jnp.dot(g, wh_ref[...], preferred_element_type=F32) + bh_ref[...]
            rr = jax.nn.sigmoid(gi[:, 0:d] + gh[:, 0:d])
            zz = jax.nn.sigmoid(gi[:, d:2 * d] + gh[:, d:2 * d])
            nn_ = jnp.tanh(gi[:, 2 * d:3 * d] + rr * gh[:, 2 * d:3 * d])
            g = (1.0 - zz) * nn_ + zz * g
        out_ref[...] = g

    return pl.pallas_call(
        body,
        out_shape=jax.ShapeDtypeStruct((1, d), F32))(
            raw, den, pw, pb.reshape(1, d), awg, awh, ab.reshape(1, 1),
            gwi, gwh, gbi.reshape(1, 3 * d), gbh.reshape(1, 3 * d))


def kernel(x, edge_index, edge_attr, params):
    p = params
    n, d_in = x.shape
    e = edge_index.shape[1]
    nhalf = n // 2
    heads, hid = p['gat1_al'].shape
    d_out = p['gat3_al'].shape[1]
    src = edge_index[0]
    dst = edge_index[1]

    dsh, dsq = _tc_dstsel(dst.reshape(1, e), n)
    dsh = dsh.reshape(2 * e)
    dsq = dsq.reshape(4 * e)

    h = _tc_linear(x, p['proj_W'], p['proj_b'], act="relu", out_w=128)
    u = _tc_linear(edge_attr, p['enet_W1'], p['enet_b1'], act="relu", nb=2048)
    kk = u.shape[1]
    w2r = p['enet_W2'].reshape(kk, d_in, d_in)
    w2r = jnp.concatenate(
        [w2r, jnp.zeros((kk, 128 - d_in, d_in), F32)], axis=1)
    b2r = _pad_rows(p['enet_b2'].reshape(d_in, d_in), 128)
    gru_wi = _pad_rows(p['gru_Wi'], 128)
    gru_wh = _pad_rows(p['gru_Wh'], 128)
    hidden = h
    for _ in range(3):
        (hs,) = _sc_gather([h], [src])
        msg = _mp_messages(u, hs, w2r, b2r)
        (agg,) = _sc_scatter([msg], dsh, n)
        h = _tc_gru(agg, hidden, gru_wi, gru_wh, p['gru_bi'], p['gru_bh'],
                    d_in)
        hidden = h

    hcur = h
    for nm, dh, pre, wpad in (("gat1", hid, "none", 128),
                              ("gat2", hid, "elu", 0),
                              ("gat3", d_out, "elu", 0)):
        w = _pad_rows(p[nm + '_W'], wpad) if wpad else p[nm + '_W']
        z, tab, mx = _tc_gat_z(hcur, w, p[nm + '_al'], p[nm + '_ar'],
                               heads, dh, pre=pre)
        a_g, b_g, zg = _sc_gather([tab, tab, z], [src, dst, src])
        ex = _tc_gat_ex(a_g, b_g, mx, heads)
        (den,) = _sc_scatter([ex], dsh, n)
        (bg,) = _sc_gather([den], [dst])
        pays = _tc_gat_pay(zg, ex, bg, heads, dh)
        outs = _sc_scatter(list(pays), dsh, n)
        hcur = outs[0] if len(outs) == 1 else jnp.concatenate(outs, axis=1)

    raw, den_prev = hcur, None
    for nm, wpad in (("ga1", 128), ("ga2", 0), ("ga3", 0)):
        aw = p[nm + '_aw']
        m = aw.shape[0] // 2
        w = _pad_rows(p[nm + '_W'], wpad) if wpad else p[nm + '_W']
        t, tab, mx = _tc_ga_t(raw, w, p[nm + '_b'], aw[:m], aw[m:],
                              p[nm + '_ab'][0], den=den_prev)
        a_g, b_g, tg = _sc_gather([tab, tab, t], [src, dst, src])
        expays = _tc_ga_expay(a_g, b_g, mx, tg)
        souts = _sc_scatter(list(expays), dsq, n, parts=4)
        den_prev = souts[0]
        raw = (souts[1] if len(souts) == 2
               else jnp.concatenate(souts[1:], axis=1))

    attw = p['ro_attW']
    return _tc_readout(raw, den_prev, p['ro_pW'], p['ro_pb'],
                       attw[:d_out], attw[d_out:], p['ro_attb'][0],
                       p['ro_gWi'], p['ro_gWh'], p['ro_gbi'], p['ro_gbh'],
                       d_out)


# speedup vs baseline: 7.2665x; 1.0807x over previous
import jax
import jax.numpy as jnp
from jax import lax
from jax.experimental import pallas as pl
from jax.experimental.pallas import tpu as pltpu
from jax.experimental.pallas import tpu_sc as plsc

F32 = jnp.float32
NC, NS = 2, 16
NW = NC * NS
BIG = 1e9
EPS = 1e-16


def _sc_mesh():
    return plsc.VectorSubcoreMesh(core_axis_name="c", subcore_axis_name="s")


def _pad_rows(w, rows):
    return jnp.concatenate(
        [w, jnp.zeros((rows - w.shape[0],) + w.shape[1:], F32)], axis=0)


def _sc_gather(tables, idxs):
    e = idxs[0].shape[0]
    ns = len(tables)
    epw = e // NW
    widths = [int(t.shape[1]) for t in tables]
    chs = [64 if w <= 256 else 32 for w in widths]
    out_type = tuple(jax.ShapeDtypeStruct((e, w), F32) for w in widths)
    scratch = []
    for w, ch in zip(widths, chs):
        scratch.append(pltpu.VMEM((2, ch), jnp.int32))
        scratch.append(pltpu.VMEM((2, ch, w), F32))
        scratch.append(pltpu.SemaphoreType.DMA)
        scratch.append(pltpu.SemaphoreType.DMA)

    def body(*refs):
        tabs = refs[:ns]
        idr = refs[ns:2 * ns]
        outs = refs[2 * ns:3 * ns]
        scr = refs[3 * ns:]
        wid = lax.axis_index("s") * NC + lax.axis_index("c")
        base = wid * epw
        for i in range(ns):
            iv, rv = scr[4 * i], scr[4 * i + 1]
            sems = (scr[4 * i + 2], scr[4 * i + 3])
            ch = chs[i]
            nch = epw // ch
            handles = [None, None]
            pltpu.sync_copy(idr[i].at[pl.ds(pl.multiple_of(base, 8), ch)],
                            iv.at[0])
            handles[0] = pltpu.async_copy(tabs[i].at[iv.at[0]], rv.at[0],
                                          sems[0])
            for k in range(nch):
                cur = k & 1
                if k + 1 < nch:
                    nxt = 1 - cur
                    off1 = pl.multiple_of(base + (k + 1) * ch, 8)
                    pltpu.sync_copy(idr[i].at[pl.ds(off1, ch)], iv.at[nxt])
                    handles[nxt] = pltpu.async_copy(
                        tabs[i].at[iv.at[nxt]], rv.at[nxt], sems[nxt])
                handles[cur].wait()
                off = pl.multiple_of(base + k * ch, 8)
                pltpu.sync_copy(rv.at[cur], outs[i].at[pl.ds(off, ch)])

    return pl.kernel(body, out_type=out_type, mesh=_sc_mesh(),
                     scratch_types=tuple(scratch))(*tables, *idxs)


def _sc_scatter(payloads, dst_sel_flat, n, parts=2):
    e = payloads[0].shape[0]
    ns = len(payloads)
    phases = parts // NC
    nq = n // parts
    acc_rows = nq + 128
    stripe = acc_rows // NS
    orows = nq // NS
    ept = e // NS
    ch = 64
    widths = [int(p.shape[1]) for p in payloads]
    out_type = tuple(jax.ShapeDtypeStruct((n, w), F32) for w in widths)
    scratch = [pltpu.VMEM((2, ch), jnp.int32)]
    for w in widths:
        scratch.append(pltpu.VMEM((8, w), F32))
        scratch.append(pltpu.VMEM((2, ch, w), F32))
        scratch.append(pltpu.VMEM_SHARED((acc_rows, w), F32))
        scratch.append(pltpu.SemaphoreType.DMA)
        scratch.append(pltpu.SemaphoreType.DMA)

    def body(dsr, *refs):
        pays = refs[:ns]
        outs = refs[ns:2 * ns]
        scr = refs[2 * ns:]
        iv = scr[0]
        c = lax.axis_index("c")
        s = lax.axis_index("s")
        nch = ept // ch
        for q in range(phases):
            for i in range(ns):
                zb, acc = scr[1 + 5 * i], scr[3 + 5 * i]
                if q == 0:
                    for r in range(8):
                        for j in range(widths[i] // 16):
                            zb[r, pl.ds(j * 16, 16)] = jnp.zeros((16,), F32)
                for j in range(stripe // 8):
                    zo = pl.multiple_of(s * stripe + j * 8, 8)
                    pltpu.sync_copy(zb, acc.at[pl.ds(zo, 8)])
            plsc.subcore_barrier()
            handles = [[None, None] for _ in range(ns)]
            for k in range(nch):
                cur = k & 1
                ioff = pl.multiple_of(
                    (phases * c + q) * e + s * ept + k * ch, 8)
                eoff = pl.multiple_of(s * ept + k * ch, 8)
                for i in range(ns):
                    if handles[i][cur] is not None:
                        handles[i][cur].wait()
                pltpu.sync_copy(dsr.at[pl.ds(ioff, ch)], iv.at[cur])
                for i in range(ns):
                    pv, acc = scr[2 + 5 * i], scr[3 + 5 * i]
                    sem = scr[4 + 5 * i + cur]
                    pltpu.sync_copy(pays[i].at[pl.ds(eoff, ch)], pv.at[cur])
                    handles[i][cur] = pltpu.async_copy(
                        pv.at[cur], acc.at[iv.at[cur]], sem, add=True)
            for i in range(ns):
                for par in range(2):
                    if handles[i][par] is not None:
                        handles[i][par].wait()
            plsc.subcore_barrier()
            so = pl.multiple_of(s * orows, 8)
            oo = pl.multiple_of((phases * c + q) * nq + s * orows, 8)
            for i in range(ns):
                acc = scr[3 + 5 * i]
                pltpu.sync_copy(acc.at[pl.ds(so, orows)],
                                outs[i].at[pl.ds(oo, orows)])
            plsc.subcore_barrier()

    return pl.kernel(body, out_type=out_type, mesh=_sc_mesh(),
                     scratch_types=tuple(scratch))(dst_sel_flat, *payloads)


def _tc_dstsel(dst2d, n):
    e = dst2d.shape[1]
    eb = 2048

    def body(d_ref, oh_ref, oq_ref):
        d = d_ref[...]
        for parts, o_ref in ((2, oh_ref), (4, oq_ref)):
            r = n // parts
            sels = []
            for q in range(parts):
                dq = d - q * r
                sels.append(jnp.where((dq >= 0) & (dq < r), dq, r))
            o_ref[...] = jnp.concatenate(sels, axis=0)

    return pl.pallas_call(
        body, grid=(e // eb,),
        in_specs=[pl.BlockSpec((1, eb), lambda i: (0, i))],
        out_specs=[pl.BlockSpec((2, eb), lambda i: (0, i)),
                   pl.BlockSpec((4, eb), lambda i: (0, i))],
        out_shape=[jax.ShapeDtypeStruct((2, e), jnp.int32),
                   jax.ShapeDtypeStruct((4, e), jnp.int32)])(dst2d)


def _pre_act(xb, pre, d=None):
    if d is not None:
        xb = xb / (d[:, 0:1] + EPS)
    if pre == "relu":
        xb = jnp.maximum(xb, 0.0)
    elif pre == "elu":
        xb = jnp.where(xb > 0, xb, jnp.exp(jnp.minimum(xb, 0.0)) - 1.0)
    return xb


def _tc_linear(x, w, b, act="none", out_w=None, nb=1024):
    n, kdim = x.shape
    m = w.shape[-1]
    mo = m if out_w is None else out_w

    def body(x_ref, w_ref, b_ref, out_ref):
        y = jnp.dot(x_ref[...], w_ref[...], preferred_element_type=F32)
        y = y + b_ref[...]
        if act == "relu":
            y = jnp.maximum(y, 0.0)
        if mo > m:
            y = jnp.concatenate(
                [y, jnp.zeros((y.shape[0], mo - m), F32)], axis=1)
        out_ref[...] = y

    return pl.pallas_call(
        body, grid=(n // nb,),
        in_specs=[pl.BlockSpec((nb, kdim), lambda i: (i, 0)),
                  pl.BlockSpec((kdim, m), lambda i: (0, 0)),
                  pl.BlockSpec((1, m), lambda i: (0, 0))],
        out_specs=pl.BlockSpec((nb, mo), lambda i: (i, 0)),
        out_shape=jax.ShapeDtypeStruct((n, mo), F32))(x, w, b.reshape(1, m))


def _mp_msg_body(u_ref, hs_ref, w2r_ref, b2r_ref, out_ref):
    u = u_ref[...]
    hs = hs_ref[...]
    d = b2r_ref.shape[1]
    acc = jnp.dot(hs, b2r_ref[...], preferred_element_type=F32)
    for k in range(u.shape[1]):
        acc += u[:, k:k + 1] * jnp.dot(hs, w2r_ref[k],
                                       preferred_element_type=F32)
    acc = jnp.concatenate([acc, jnp.zeros((acc.shape[0], 128 - d), F32)],
                          axis=1)
    out_ref[...] = acc


def _mp_messages(u, hs, w2r, b2r):
    e = hs.shape[0]
    d = b2r.shape[1]
    kk = u.shape[1]
    eb = 2048
    return pl.pallas_call(
        _mp_msg_body,
        grid=(e // eb,),
        in_specs=[
            pl.BlockSpec((eb, kk), lambda i: (i, 0)),
            pl.BlockSpec((eb, 128), lambda i: (i, 0)),
            pl.BlockSpec((kk, 128, d), lambda i: (0, 0, 0)),
            pl.BlockSpec((128, d), lambda i: (0, 0)),
        ],
        out_specs=pl.BlockSpec((eb, 128), lambda i: (i, 0)),
        out_shape=jax.ShapeDtypeStruct((e, 128), F32))(u, hs, w2r, b2r)


def _tc_gru(agg, hidden, wi, wh, bi, bh, d, nb=1024):
    n = hidden.shape[0]

    def body(a_ref, h_ref, wi_ref, wh_ref, bi_ref, bh_ref, out_ref):
        m = jnp.maximum(a_ref[...], 0.0)
        hp = h_ref[...]
        gi = jnp.dot(m, wi_ref[...], preferred_element_type=F32) + bi_ref[...]
        gh = jnp.dot(hp, wh_ref[...], preferred_element_type=F32) + bh_ref[...]
        r = jax.nn.sigmoid(gi[:, 0:d] + gh[:, 0:d])
        z = jax.nn.sigmoid(gi[:, d:2 * d] + gh[:, d:2 * d])
        nn_ = jnp.tanh(gi[:, 2 * d:3 * d] + r * gh[:, 2 * d:3 * d])
        y = (1.0 - z) * nn_ + z * hp[:, 0:d]
        out_ref[...] = jnp.concatenate(
            [y, jnp.zeros((y.shape[0], 128 - d), F32)], axis=1)

    return pl.pallas_call(
        body, grid=(n // nb,),
        in_specs=[
            pl.BlockSpec((nb, 128), lambda i: (i, 0)),
            pl.BlockSpec((nb, 128), lambda i: (i, 0)),
            pl.BlockSpec((128, 3 * d), lambda i: (0, 0)),
            pl.BlockSpec((128, 3 * d), lambda i: (0, 0)),
            pl.BlockSpec((1, 3 * d), lambda i: (0, 0)),
            pl.BlockSpec((1, 3 * d), lambda i: (0, 0)),
        ],
        out_specs=pl.BlockSpec((nb, 128), lambda i: (i, 0)),
        out_shape=jax.ShapeDtypeStruct((n, 128), F32))(
            agg, hidden, wi, wh, bi.reshape(1, 3 * d), bh.reshape(1, 3 * d))


def _tc_gat_z(x, w, al, ar, heads, dh, pre="none", nb=1024):
    n, kdim = x.shape
    m = heads * dh

    def body(x_ref, w_ref, al_ref, ar_ref, z_ref, tab_ref, mx_ref):
        i = pl.program_id(0)
        xb = _pre_act(x_ref[...], pre)
        z = jnp.dot(xb, w_ref[...], preferred_element_type=F32)
        z_ref[...] = z
        els, ers = [], []
        for h in range(heads):
            zh = z[:, h * dh:(h + 1) * dh]
            els.append(jnp.sum(zh * al_ref[h:h + 1, :], axis=1, keepdims=True))
            ers.append(jnp.sum(zh * ar_ref[h:h + 1, :], axis=1, keepdims=True))
        el = jnp.concatenate(els, axis=1)
        er = jnp.concatenate(ers, axis=1)
        pad = jnp.zeros((xb.shape[0], 64 - heads), F32)
        tab_ref[...] = jnp.concatenate([el, pad, er, pad], axis=1)
        cur = jnp.concatenate(
            [jnp.max(el, axis=0, keepdims=True),
             jnp.max(er, axis=0, keepdims=True),
             jnp.full((1, 16 - 2 * heads), -BIG, F32)], axis=1)

        @pl.when(i == 0)
        def _():
            mx_ref[...] = cur

        @pl.when(i > 0)
        def _():
            mx_ref[...] = jnp.maximum(mx_ref[...], cur)

    return pl.pallas_call(
        body, grid=(n // nb,),
        in_specs=[
            pl.BlockSpec((nb, kdim), lambda i: (i, 0)),
            pl.BlockSpec((kdim, m), lambda i: (0, 0)),
            pl.BlockSpec((heads, dh), lambda i: (0, 0)),
            pl.BlockSpec((heads, dh), lambda i: (0, 0)),
        ],
        out_specs=[
            pl.BlockSpec((nb, m), lambda i: (i, 0)),
            pl.BlockSpec((nb, 128), lambda i: (i, 0)),
            pl.BlockSpec((1, 16), lambda i: (0, 0)),
        ],
        out_shape=[
            jax.ShapeDtypeStruct((n, m), F32),
            jax.ShapeDtypeStruct((n, 128), F32),
            jax.ShapeDtypeStruct((1, 16), F32),
        ])(x, w, al, ar)


def _tc_gat_ex(a, b, mx, heads, eb=2048):
    e = a.shape[0]

    def body(a_ref, b_ref, mx_ref, out_ref):
        m = mx_ref[...]
        s = m[:, 0:heads] + m[:, heads:2 * heads]
        s = jnp.maximum(s, 0.2 * s)
        ee = a_ref[..., 0:heads] + b_ref[..., 64:64 + heads]
        ee = jnp.maximum(ee, 0.2 * ee)
        ex = jnp.exp(ee - s)
        out_ref[...] = jnp.concatenate(
            [ex, jnp.zeros((ex.shape[0], 128 - heads), F32)], axis=1)

    return pl.pallas_call(
        body, grid=(e // eb,),
        in_specs=[
            pl.BlockSpec((eb, 128), lambda i: (i, 0)),
            pl.BlockSpec((eb, 128), lambda i: (i, 0)),
            pl.BlockSpec((1, 16), lambda i: (0, 0)),
        ],
        out_specs=pl.BlockSpec((eb, 128), lambda i: (i, 0)),
        out_shape=jax.ShapeDtypeStruct((e, 128), F32))(a, b, mx)


def _tc_binv(den, nb=1024):
    n = den.shape[0]

    def body(d_ref, o_ref):
        o_ref[...] = 1.0 / (d_ref[...] + EPS)

    return pl.pallas_call(
        body, grid=(n // nb,),
        in_specs=[pl.BlockSpec((nb, 128), lambda i: (i, 0))],
        out_specs=pl.BlockSpec((nb, 128), lambda i: (i, 0)),
        out_shape=jax.ShapeDtypeStruct((n, 128), F32))(den)


def _tc_gat_pay(zg, ex, bg, heads, dh, eb=2048):
    e, m = zg.shape
    mo = max(dh, 128)
    nch = mo // 128

    def body(z_ref, e_ref, b_ref, *outs):
        w = (e_ref[..., 0:heads] / (b_ref[..., 0:heads] + EPS)
             * (1.0 / heads))
        z = z_ref[...]
        acc = w[:, 0:1] * z[:, 0:dh]
        for h in range(1, heads):
            acc += w[:, h:h + 1] * z[:, h * dh:(h + 1) * dh]
        if mo > dh:
            acc = jnp.concatenate(
                [acc, jnp.zeros((acc.shape[0], mo - dh), F32)], axis=1)
        for j in range(nch):
            outs[j][...] = acc[:, j * 128:(j + 1) * 128]

    return pl.pallas_call(
        body, grid=(e // eb,),
        in_specs=[
            pl.BlockSpec((eb, m), lambda i: (i, 0)),
            pl.BlockSpec((eb, 128), lambda i: (i, 0)),
            pl.BlockSpec((eb, 128), lambda i: (i, 0)),
        ],
        out_specs=[pl.BlockSpec((eb, 128), lambda i: (i, 0))] * nch,
        out_shape=[jax.ShapeDtypeStruct((e, 128), F32)] * nch)(zg, ex, bg)


def _tc_ga_t(x, w, b, awl, awr, ab, pre="none", den=None, nb=1024):
    n, kdim = x.shape
    m = w.shape[1]
    mo = max(m, 128)
    ins = [x, w, b.reshape(1, m), awl, awr, ab.reshape(1, 1)]
    specs = [pl.BlockSpec((nb, kdim), lambda i: (i, 0)),
             pl.BlockSpec((kdim, m), lambda i: (0, 0)),
             pl.BlockSpec((1, m), lambda i: (0, 0)),
             pl.BlockSpec((m, 1), lambda i: (0, 0)),
             pl.BlockSpec((m, 1), lambda i: (0, 0)),
             pl.BlockSpec((1, 1), lambda i: (0, 0))]
    if den is not None:
        ins.append(den)
        specs.append(pl.BlockSpec((nb, 128), lambda i: (i, 0)))

    def body(x_ref, w_ref, b_ref, awl_ref, awr_ref, ab_ref, *rest):
        t_ref, tab_ref, mx_ref = rest[-3:]
        d = rest[0][...] if den is not None else None
        i = pl.program_id(0)
        xb = _pre_act(x_ref[...], pre, d)
        t = jnp.dot(xb, w_ref[...], preferred_element_type=F32) + b_ref[...]
        tl = jnp.dot(t, awl_ref[...], preferred_element_type=F32) + ab_ref[...]
        tr = jnp.dot(t, awr_ref[...], preferred_element_type=F32)
        if mo > m:
            t = jnp.concatenate(
                [t, jnp.zeros((t.shape[0], mo - m), F32)], axis=1)
        t_ref[...] = t
        pad = jnp.zeros((xb.shape[0], 63), F32)
        tab_ref[...] = jnp.concatenate([tl, pad, tr, pad], axis=1)
        cur = jnp.concatenate(
            [jnp.max(tl, axis=0, keepdims=True),
             jnp.max(tr, axis=0, keepdims=True),
             jnp.full((1, 14), -BIG, F32)], axis=1)

        @pl.when(i == 0)
        def _():
            mx_ref[...] = cur

        @pl.when(i > 0)
        def _():
            mx_ref[...] = jnp.maximum(mx_ref[...], cur)

    return pl.pallas_call(
        body, grid=(n // nb,), in_specs=specs,
        out_specs=[
            pl.BlockSpec((nb, mo), lambda i: (i, 0)),
            pl.BlockSpec((nb, 128), lambda i: (i, 0)),
            pl.BlockSpec((1, 16), lambda i: (0, 0)),
        ],
        out_shape=[
            jax.ShapeDtypeStruct((n, mo), F32),
            jax.ShapeDtypeStruct((n, 128), F32),
            jax.ShapeDtypeStruct((1, 16), F32),
        ])(*ins)


def _tc_ga_expay(a, b, mx, tg, eb=2048):
    e = a.shape[0]
    m = tg.shape[1]
    nch = m // 128

    def body(a_ref, b_ref, mx_ref, t_ref, ex_ref, *pouts):
        mxv = mx_ref[...]
        s = mxv[:, 0:1] + mxv[:, 1:2]
        s = jnp.maximum(s, 0.2 * s)
        ee = a_ref[..., 0:1] + b_ref[..., 64:65]
        ee = jnp.maximum(ee, 0.2 * ee)
        ex = jnp.exp(ee - s)
        ex_ref[...] = jnp.concatenate(
            [ex, jnp.zeros((ex.shape[0], 127), F32)], axis=1)
        pay = ex * t_ref[...]
        for j in range(nch):
            pouts[j][...] = pay[:, j * 128:(j + 1) * 128]

    return pl.pallas_call(
        body, grid=(e // eb,),
        in_specs=[
            pl.BlockSpec((eb, 128), lambda i: (i, 0)),
            pl.BlockSpec((eb, 128), lambda i: (i, 0)),
            pl.BlockSpec((1, 16), lambda i: (0, 0)),
            pl.BlockSpec((eb, m), lambda i: (i, 0)),
        ],
        out_specs=[pl.BlockSpec((eb, 128), lambda i: (i, 0))] * (1 + nch),
        out_shape=[jax.ShapeDtypeStruct((e, 128), F32)] * (1 + nch))(
            a, b, mx, tg)


def _tc_readout(raw, den, pw, pb, awg, awh, ab, gwi, gwh, gbi, gbh, d):
    n = raw.shape[0]

    def body(r_ref, d_ref, pw_ref, pb_ref, ag_ref, ah_ref, ab_ref,
             wi_ref, wh_ref, bi_ref, bh_ref, out_ref):
        h = r_ref[..., 0:d] / (d_ref[:, 0:1] + EPS)
        hv = jnp.dot(h, pw_ref[...], preferred_element_type=F32) + pb_ref[...]
        lg_h = jnp.dot(h, ah_ref[...], preferred_element_type=F32)
        g = jnp.sum(h, axis=0, keepdims=True)
        for _ in range(3):
            gl = jnp.dot(g, ag_ref[...], preferred_element_type=F32) + ab_ref[...]
            logits = lg_h + gl
            logits = jnp.maximum(logits, 0.01 * logits)
            mxl = jnp.max(logits, axis=0, keepdims=True)
            aa = jnp.exp(logits - mxl)
            aa = aa / jnp.sum(aa, axis=0, keepdims=True)
            context = jnp.sum(aa * hv, axis=0, keepdims=True)
            context = jnp.where(context > 0, context,
                                jnp.exp(jnp.minimum(context, 0.0)) - 1.0)
            gi = jnp.dot(context, wi_ref[...], preferred_element_type=F32) + bi_ref[...]
            gh = jnp.dot(g, wh_ref[...], preferred_element_type=F32) + bh_ref[...]
            rr = jax.nn.sigmoid(gi[:, 0:d] + gh[:, 0:d])
            zz = jax.nn.sigmoid(gi[:, d:2 * d] + gh[:, d:2 * d])
            nn_ = jnp.tanh(gi[:, 2 * d:3 * d] + rr * gh[:, 2 * d:3 * d])
            g = (1.0 - zz) * nn_ + zz * g
        out_ref[...] = g

    return pl.pallas_call(
        body,
        out_shape=jax.ShapeDtypeStruct((1, d), F32))(
            raw, den, pw, pb.reshape(1, d), awg, awh, ab.reshape(1, 1),
            gwi, gwh, gbi.reshape(1, 3 * d), gbh.reshape(1, 3 * d))


def kernel(x, edge_index, edge_attr, params):
    p = params
    n, d_in = x.shape
    e = edge_index.shape[1]
    nhalf = n // 2
    heads, hid = p['gat1_al'].shape
    d_out = p['gat3_al'].shape[1]
    src = edge_index[0]
    dst = edge_index[1]

    dsh, dsq = _tc_dstsel(dst.reshape(1, e), n)
    dsh = dsh.reshape(2 * e)
    dsq = dsq.reshape(4 * e)

    h = _tc_linear(x, p['proj_W'], p['proj_b'], act="relu", out_w=128)
    u = _tc_linear(edge_attr, p['enet_W1'], p['enet_b1'], act="relu", nb=2048)
    kk = u.shape[1]
    w2r = p['enet_W2'].reshape(kk, d_in, d_in)
    w2r = jnp.concatenate(
        [w2r, jnp.zeros((kk, 128 - d_in, d_in), F32)], axis=1)
    b2r = _pad_rows(p['enet_b2'].reshape(d_in, d_in), 128)
    gru_wi = _pad_rows(p['gru_Wi'], 128)
    gru_wh = _pad_rows(p['gru_Wh'], 128)
    hidden = h
    for _ in range(3):
        (hs,) = _sc_gather([h], [src])
        msg = _mp_messages(u, hs, w2r, b2r)
        (agg,) = _sc_scatter([msg], dsh, n)
        h = _tc_gru(agg, hidden, gru_wi, gru_wh, p['gru_bi'], p['gru_bh'],
                    d_in)
        hidden = h

    hcur = h
    for nm, dh, pre, wpad in (("gat1", hid, "none", 128),
                              ("gat2", hid, "elu", 0),
                              ("gat3", d_out, "elu", 0)):
        w = _pad_rows(p[nm + '_W'], wpad) if wpad else p[nm + '_W']
        z, tab, mx = _tc_gat_z(hcur, w, p[nm + '_al'], p[nm + '_ar'],
                               heads, dh, pre=pre)
        a_g, b_g, zg = _sc_gather([tab, tab, z], [src, dst, src])
        ex = _tc_gat_ex(a_g, b_g, mx, heads)
        (den,) = _sc_scatter([ex], dsh, n)
        (bg,) = _sc_gather([den], [dst])
        pays = _tc_gat_pay(zg, ex, bg, heads, dh)
        outs = _sc_scatter(list(pays), dsh, n)
        hcur = outs[0] if len(outs) == 1 else jnp.concatenate(outs, axis=1)

    raw, den_prev = hcur, None
    for nm, wpad in (("ga1", 128), ("ga2", 0), ("ga3", 0)):
        aw = p[nm + '_aw']
        m = aw.shape[0] // 2
        w = _pad_rows(p[nm + '_W'], wpad) if wpad else p[nm + '_W']
        t, tab, mx = _tc_ga_t(raw, w, p[nm + '_b'], aw[:m], aw[m:],
                              p[nm + '_ab'][0], den=den_prev)
        a_g, b_g, tg = _sc_gather([tab, tab, t], [src, dst, src])
        expays = _tc_ga_expay(a_g, b_g, mx, tg)
        (den_prev,) = _sc_scatter([expays[0]], dsh, n)
        souts = _sc_scatter(list(expays[1:]), dsh, n)
        raw = (souts[0] if len(souts) == 1
               else jnp.concatenate(souts, axis=1))

    attw = p['ro_attW']
    return _tc_readout(raw, den_prev, p['ro_pW'], p['ro_pb'],
                       attw[:d_out], attw[d_out:], p['ro_attb'][0],
                       p['ro_gWi'], p['ro_gWh'], p['ro_gbi'], p['ro_gbh'],
                       d_out)


# speedup vs baseline: 7.3491x; 1.0114x over previous
import jax
import jax.numpy as jnp
from jax import lax
from jax.experimental import pallas as pl
from jax.experimental.pallas import tpu as pltpu
from jax.experimental.pallas import tpu_sc as plsc

F32 = jnp.float32
NC, NS = 2, 16
NW = NC * NS
BIG = 1e9
EPS = 1e-16


def _sc_mesh():
    return plsc.VectorSubcoreMesh(core_axis_name="c", subcore_axis_name="s")


def _pad_rows(w, rows):
    return jnp.concatenate(
        [w, jnp.zeros((rows - w.shape[0],) + w.shape[1:], F32)], axis=0)


def _sc_gather(tables, idxs):
    e = idxs[0].shape[0]
    ns = len(tables)
    epw = e // NW
    widths = [int(t.shape[1]) for t in tables]
    chs = [64 if w <= 256 else 32 for w in widths]
    out_type = tuple(jax.ShapeDtypeStruct((e, w), F32) for w in widths)
    scratch = []
    for w, ch in zip(widths, chs):
        scratch.append(pltpu.VMEM((2, ch), jnp.int32))
        scratch.append(pltpu.VMEM((2, ch, w), F32))
        scratch.append(pltpu.SemaphoreType.DMA)
        scratch.append(pltpu.SemaphoreType.DMA)

    def body(*refs):
        tabs = refs[:ns]
        idr = refs[ns:2 * ns]
        outs = refs[2 * ns:3 * ns]
        scr = refs[3 * ns:]
        wid = lax.axis_index("s") * NC + lax.axis_index("c")
        base = wid * epw
        for i in range(ns):
            iv, rv = scr[4 * i], scr[4 * i + 1]
            sems = (scr[4 * i + 2], scr[4 * i + 3])
            ch = chs[i]
            nch = epw // ch
            handles = [None, None]
            pltpu.sync_copy(idr[i].at[pl.ds(pl.multiple_of(base, 8), ch)],
                            iv.at[0])
            handles[0] = pltpu.async_copy(tabs[i].at[iv.at[0]], rv.at[0],
                                          sems[0])
            for k in range(nch):
                cur = k & 1
                if k + 1 < nch:
                    nxt = 1 - cur
                    off1 = pl.multiple_of(base + (k + 1) * ch, 8)
                    pltpu.sync_copy(idr[i].at[pl.ds(off1, ch)], iv.at[nxt])
                    handles[nxt] = pltpu.async_copy(
                        tabs[i].at[iv.at[nxt]], rv.at[nxt], sems[nxt])
                handles[cur].wait()
                off = pl.multiple_of(base + k * ch, 8)
                pltpu.sync_copy(rv.at[cur], outs[i].at[pl.ds(off, ch)])

    return pl.kernel(body, out_type=out_type, mesh=_sc_mesh(),
                     scratch_types=tuple(scratch))(*tables, *idxs)


def _sc_scatter(payloads, dst_sel_flat, n, parts=2):
    e = payloads[0].shape[0]
    ns = len(payloads)
    phases = parts // NC
    nq = n // parts
    acc_rows = nq + 128
    stripe = acc_rows // NS
    orows = nq // NS
    ept = e // NS
    ch = 64
    widths = [int(p.shape[1]) for p in payloads]
    out_type = tuple(jax.ShapeDtypeStruct((n, w), F32) for w in widths)
    scratch = [pltpu.VMEM((2, ch), jnp.int32)]
    for w in widths:
        scratch.append(pltpu.VMEM((8, w), F32))
        scratch.append(pltpu.VMEM((2, ch, w), F32))
        scratch.append(pltpu.VMEM_SHARED((acc_rows, w), F32))
        scratch.append(pltpu.SemaphoreType.DMA)
        scratch.append(pltpu.SemaphoreType.DMA)

    def body(dsr, *refs):
        pays = refs[:ns]
        outs = refs[ns:2 * ns]
        scr = refs[2 * ns:]
        iv = scr[0]
        c = lax.axis_index("c")
        s = lax.axis_index("s")
        nch = ept // ch
        for q in range(phases):
            for i in range(ns):
                zb, acc = scr[1 + 5 * i], scr[3 + 5 * i]
                if q == 0:
                    for r in range(8):
                        for j in range(widths[i] // 16):
                            zb[r, pl.ds(j * 16, 16)] = jnp.zeros((16,), F32)
                for j in range(stripe // 8):
                    zo = pl.multiple_of(s * stripe + j * 8, 8)
                    pltpu.sync_copy(zb, acc.at[pl.ds(zo, 8)])
            plsc.subcore_barrier()
            handles = [[None, None] for _ in range(ns)]
            for k in range(nch):
                cur = k & 1
                ioff = pl.multiple_of(
                    (phases * c + q) * e + s * ept + k * ch, 8)
                eoff = pl.multiple_of(s * ept + k * ch, 8)
                for i in range(ns):
                    if handles[i][cur] is not None:
                        handles[i][cur].wait()
                pltpu.sync_copy(dsr.at[pl.ds(ioff, ch)], iv.at[cur])
                for i in range(ns):
                    pv, acc = scr[2 + 5 * i], scr[3 + 5 * i]
                    sem = scr[4 + 5 * i + cur]
                    pltpu.sync_copy(pays[i].at[pl.ds(eoff, ch)], pv.at[cur])
                    handles[i][cur] = pltpu.async_copy(
                        pv.at[cur], acc.at[iv.at[cur]], sem, add=True)
            for i in range(ns):
                for par in range(2):
                    if handles[i][par] is not None:
                        handles[i][par].wait()
            plsc.subcore_barrier()
            so = pl.multiple_of(s * orows, 8)
            oo = pl.multiple_of((phases * c + q) * nq + s * orows, 8)
            for i in range(ns):
                acc = scr[3 + 5 * i]
                pltpu.sync_copy(acc.at[pl.ds(so, orows)],
                                outs[i].at[pl.ds(oo, orows)])
            plsc.subcore_barrier()

    return pl.kernel(body, out_type=out_type, mesh=_sc_mesh(),
                     scratch_types=tuple(scratch))(dst_sel_flat, *payloads)


def _tc_dstsel(dst2d, n):
    e = dst2d.shape[1]
    eb = 2048

    def body(d_ref, oh_ref, oq_ref):
        d = d_ref[...]
        for parts, o_ref in ((2, oh_ref), (4, oq_ref)):
            r = n // parts
            sels = []
            for q in range(parts):
                dq = d - q * r
                sels.append(jnp.where((dq >= 0) & (dq < r), dq, r))
            o_ref[...] = jnp.concatenate(sels, axis=0)

    return pl.pallas_call(
        body, grid=(e // eb,),
        in_specs=[pl.BlockSpec((1, eb), lambda i: (0, i))],
        out_specs=[pl.BlockSpec((2, eb), lambda i: (0, i)),
                   pl.BlockSpec((4, eb), lambda i: (0, i))],
        out_shape=[jax.ShapeDtypeStruct((2, e), jnp.int32),
                   jax.ShapeDtypeStruct((4, e), jnp.int32)])(dst2d)


def _pre_act(xb, pre, d=None):
    if d is not None:
        xb = xb / (d[:, 0:1] + EPS)
    if pre == "relu":
        xb = jnp.maximum(xb, 0.0)
    elif pre == "elu":
        xb = jnp.where(xb > 0, xb, jnp.exp(jnp.minimum(xb, 0.0)) - 1.0)
    return xb


def _tc_linear(x, w, b, act="none", out_w=None, nb=1024):
    n, kdim = x.shape
    m = w.shape[-1]
    mo = m if out_w is None else out_w

    def body(x_ref, w_ref, b_ref, out_ref):
        y = jnp.dot(x_ref[...], w_ref[...], preferred_element_type=F32)
        y = y + b_ref[...]
        if act == "relu":
            y = jnp.maximum(y, 0.0)
        if mo > m:
            y = jnp.concatenate(
                [y, jnp.zeros((y.shape[0], mo - m), F32)], axis=1)
        out_ref[...] = y

    return pl.pallas_call(
        body, grid=(n // nb,),
        in_specs=[pl.BlockSpec((nb, kdim), lambda i: (i, 0)),
                  pl.BlockSpec((kdim, m), lambda i: (0, 0)),
                  pl.BlockSpec((1, m), lambda i: (0, 0))],
        out_specs=pl.BlockSpec((nb, mo), lambda i: (i, 0)),
        out_shape=jax.ShapeDtypeStruct((n, mo), F32))(x, w, b.reshape(1, m))


def _mp_msg_body(u_ref, hs_ref, w2r_ref, b2r_ref, out_ref):
    u = u_ref[...]
    hs = hs_ref[...]
    d = b2r_ref.shape[1]
    acc = jnp.dot(hs, b2r_ref[...], preferred_element_type=F32)
    for k in range(u.shape[1]):
        acc += u[:, k:k + 1] * jnp.dot(hs, w2r_ref[k],
                                       preferred_element_type=F32)
    acc = jnp.concatenate([acc, jnp.zeros((acc.shape[0], 128 - d), F32)],
                          axis=1)
    out_ref[...] = acc


def _mp_messages(u, hs, w2r, b2r):
    e = hs.shape[0]
    d = b2r.shape[1]
    kk = u.shape[1]
    eb = 2048
    return pl.pallas_call(
        _mp_msg_body,
        grid=(e // eb,),
        in_specs=[
            pl.BlockSpec((eb, kk), lambda i: (i, 0)),
            pl.BlockSpec((eb, 128), lambda i: (i, 0)),
            pl.BlockSpec((kk, 128, d), lambda i: (0, 0, 0)),
            pl.BlockSpec((128, d), lambda i: (0, 0)),
        ],
        out_specs=pl.BlockSpec((eb, 128), lambda i: (i, 0)),
        out_shape=jax.ShapeDtypeStruct((e, 128), F32))(u, hs, w2r, b2r)


def _tc_gru(agg, hidden, wi, wh, bi, bh, d, nb=1024):
    n = hidden.shape[0]

    def body(a_ref, h_ref, wi_ref, wh_ref, bi_ref, bh_ref, out_ref):
        m = jnp.maximum(a_ref[...], 0.0)
        hp = h_ref[...]
        gi = jnp.dot(m, wi_ref[...], preferred_element_type=F32) + bi_ref[...]
        gh = jnp.dot(hp, wh_ref[...], preferred_element_type=F32) + bh_ref[...]
        r = jax.nn.sigmoid(gi[:, 0:d] + gh[:, 0:d])
        z = jax.nn.sigmoid(gi[:, d:2 * d] + gh[:, d:2 * d])
        nn_ = jnp.tanh(gi[:, 2 * d:3 * d] + r * gh[:, 2 * d:3 * d])
        y = (1.0 - z) * nn_ + z * hp[:, 0:d]
        out_ref[...] = jnp.concatenate(
            [y, jnp.zeros((y.shape[0], 128 - d), F32)], axis=1)

    return pl.pallas_call(
        body, grid=(n // nb,),
        in_specs=[
            pl.BlockSpec((nb, 128), lambda i: (i, 0)),
            pl.BlockSpec((nb, 128), lambda i: (i, 0)),
            pl.BlockSpec((128, 3 * d), lambda i: (0, 0)),
            pl.BlockSpec((128, 3 * d), lambda i: (0, 0)),
            pl.BlockSpec((1, 3 * d), lambda i: (0, 0)),
            pl.BlockSpec((1, 3 * d), lambda i: (0, 0)),
        ],
        out_specs=pl.BlockSpec((nb, 128), lambda i: (i, 0)),
        out_shape=jax.ShapeDtypeStruct((n, 128), F32))(
            agg, hidden, wi, wh, bi.reshape(1, 3 * d), bh.reshape(1, 3 * d))


def _tc_gat_z(x, w, al, ar, heads, dh, pre="none", nb=1024):
    n, kdim = x.shape
    m = heads * dh

    def body(x_ref, w_ref, al_ref, ar_ref, zt_ref, tab_ref, mx_ref):
        i = pl.program_id(0)
        xb = _pre_act(x_ref[...], pre)
        z = jnp.dot(xb, w_ref[...], preferred_element_type=F32)
        els, ers = [], []
        for h in range(heads):
            zh = z[:, h * dh:(h + 1) * dh]
            els.append(jnp.sum(zh * al_ref[h:h + 1, :], axis=1, keepdims=True))
            ers.append(jnp.sum(zh * ar_ref[h:h + 1, :], axis=1, keepdims=True))
        el = jnp.concatenate(els, axis=1)
        er = jnp.concatenate(ers, axis=1)
        pad = jnp.zeros((xb.shape[0], 128 - heads), F32)
        zt_ref[...] = jnp.concatenate([z, el, pad], axis=1)
        tab_ref[...] = jnp.concatenate([er, pad], axis=1)
        cur = jnp.concatenate(
            [jnp.max(el, axis=0, keepdims=True),
             jnp.max(er, axis=0, keepdims=True),
             jnp.full((1, 16 - 2 * heads), -BIG, F32)], axis=1)

        @pl.when(i == 0)
        def _():
            mx_ref[...] = cur

        @pl.when(i > 0)
        def _():
            mx_ref[...] = jnp.maximum(mx_ref[...], cur)

    return pl.pallas_call(
        body, grid=(n // nb,),
        in_specs=[
            pl.BlockSpec((nb, kdim), lambda i: (i, 0)),
            pl.BlockSpec((kdim, m), lambda i: (0, 0)),
            pl.BlockSpec((heads, dh), lambda i: (0, 0)),
            pl.BlockSpec((heads, dh), lambda i: (0, 0)),
        ],
        out_specs=[
            pl.BlockSpec((nb, m + 128), lambda i: (i, 0)),
            pl.BlockSpec((nb, 128), lambda i: (i, 0)),
            pl.BlockSpec((1, 16), lambda i: (0, 0)),
        ],
        out_shape=[
            jax.ShapeDtypeStruct((n, m + 128), F32),
            jax.ShapeDtypeStruct((n, 128), F32),
            jax.ShapeDtypeStruct((1, 16), F32),
        ])(x, w, al, ar)


def _tc_gat_ex(zg, b, mx, heads, m, eb=2048):
    e = b.shape[0]
    stripe = m // 128

    def body(a_ref, b_ref, mx_ref, out_ref):
        mm = mx_ref[...]
        s = mm[:, 0:heads] + mm[:, heads:2 * heads]
        s = jnp.maximum(s, 0.2 * s)
        ee = a_ref[..., 0:heads] + b_ref[..., 0:heads]
        ee = jnp.maximum(ee, 0.2 * ee)
        ex = jnp.exp(ee - s)
        out_ref[...] = jnp.concatenate(
            [ex, jnp.zeros((ex.shape[0], 128 - heads), F32)], axis=1)

    return pl.pallas_call(
        body, grid=(e // eb,),
        in_specs=[
            pl.BlockSpec((eb, 128), lambda i: (i, stripe)),
            pl.BlockSpec((eb, 128), lambda i: (i, 0)),
            pl.BlockSpec((1, 16), lambda i: (0, 0)),
        ],
        out_specs=pl.BlockSpec((eb, 128), lambda i: (i, 0)),
        out_shape=jax.ShapeDtypeStruct((e, 128), F32))(zg, b, mx)


def _tc_binv(den, nb=1024):
    n = den.shape[0]

    def body(d_ref, o_ref):
        o_ref[...] = 1.0 / (d_ref[...] + EPS)

    return pl.pallas_call(
        body, grid=(n // nb,),
        in_specs=[pl.BlockSpec((nb, 128), lambda i: (i, 0))],
        out_specs=pl.BlockSpec((nb, 128), lambda i: (i, 0)),
        out_shape=jax.ShapeDtypeStruct((n, 128), F32))(den)


def _tc_gat_pay(zg, ex, bg, heads, dh, eb=2048):
    e = zg.shape[0]
    m = heads * dh
    mo = max(dh, 128)
    nch = mo // 128

    def body(z_ref, e_ref, b_ref, *outs):
        w = (e_ref[..., 0:heads] / (b_ref[..., 0:heads] + EPS)
             * (1.0 / heads))
        z = z_ref[...]
        acc = w[:, 0:1] * z[:, 0:dh]
        for h in range(1, heads):
            acc += w[:, h:h + 1] * z[:, h * dh:(h + 1) * dh]
        if mo > dh:
            acc = jnp.concatenate(
                [acc, jnp.zeros((acc.shape[0], mo - dh), F32)], axis=1)
        for j in range(nch):
            outs[j][...] = acc[:, j * 128:(j + 1) * 128]

    return pl.pallas_call(
        body, grid=(e // eb,),
        in_specs=[
            pl.BlockSpec((eb, m), lambda i: (i, 0)),
            pl.BlockSpec((eb, 128), lambda i: (i, 0)),
            pl.BlockSpec((eb, 128), lambda i: (i, 0)),
        ],
        out_specs=[pl.BlockSpec((eb, 128), lambda i: (i, 0))] * nch,
        out_shape=[jax.ShapeDtypeStruct((e, 128), F32)] * nch)(zg, ex, bg)


def _tc_ga_t(x, w, b, awl, awr, ab, pre="none", den=None, nb=1024):
    n, kdim = x.shape
    m = w.shape[1]
    mo = max(m, 128)
    ins = [x, w, b.reshape(1, m), awl, awr, ab.reshape(1, 1)]
    specs = [pl.BlockSpec((nb, kdim), lambda i: (i, 0)),
             pl.BlockSpec((kdim, m), lambda i: (0, 0)),
             pl.BlockSpec((1, m), lambda i: (0, 0)),
             pl.BlockSpec((m, 1), lambda i: (0, 0)),
             pl.BlockSpec((m, 1), lambda i: (0, 0)),
             pl.BlockSpec((1, 1), lambda i: (0, 0))]
    if den is not None:
        ins.append(den)
        specs.append(pl.BlockSpec((nb, 128), lambda i: (i, 0)))

    def body(x_ref, w_ref, b_ref, awl_ref, awr_ref, ab_ref, *rest):
        t_ref, tab_ref, mx_ref = rest[-3:]
        d = rest[0][...] if den is not None else None
        i = pl.program_id(0)
        xb = _pre_act(x_ref[...], pre, d)
        t = jnp.dot(xb, w_ref[...], preferred_element_type=F32) + b_ref[...]
        tl = jnp.dot(t, awl_ref[...], preferred_element_type=F32) + ab_ref[...]
        tr = jnp.dot(t, awr_ref[...], preferred_element_type=F32)
        if mo > m:
            t = jnp.concatenate(
                [t, jnp.zeros((t.shape[0], mo - m), F32)], axis=1)
        pad = jnp.zeros((xb.shape[0], 127), F32)
        t_ref[...] = jnp.concatenate([t, tl, pad], axis=1)
        tab_ref[...] = jnp.concatenate([tr, pad], axis=1)
        cur = jnp.concatenate(
            [jnp.max(tl, axis=0, keepdims=True),
             jnp.max(tr, axis=0, keepdims=True),
             jnp.full((1, 14), -BIG, F32)], axis=1)

        @pl.when(i == 0)
        def _():
            mx_ref[...] = cur

        @pl.when(i > 0)
        def _():
            mx_ref[...] = jnp.maximum(mx_ref[...], cur)

    return pl.pallas_call(
        body, grid=(n // nb,), in_specs=specs,
        out_specs=[
            pl.BlockSpec((nb, mo + 128), lambda i: (i, 0)),
            pl.BlockSpec((nb, 128), lambda i: (i, 0)),
            pl.BlockSpec((1, 16), lambda i: (0, 0)),
        ],
        out_shape=[
            jax.ShapeDtypeStruct((n, mo + 128), F32),
            jax.ShapeDtypeStruct((n, 128), F32),
            jax.ShapeDtypeStruct((1, 16), F32),
        ])(*ins)


def _tc_ga_expay(tg, b, mx, m, eb=2048):
    e = b.shape[0]
    nch = m // 128
    stripe = nch

    def body(a_ref, b_ref, mx_ref, t_ref, ex_ref, *pouts):
        mxv = mx_ref[...]
        s = mxv[:, 0:1] + mxv[:, 1:2]
        s = jnp.maximum(s, 0.2 * s)
        ee = a_ref[..., 0:1] + b_ref[..., 0:1]
        ee = jnp.maximum(ee, 0.2 * ee)
        ex = jnp.exp(ee - s)
        ex_ref[...] = jnp.concatenate(
            [ex, jnp.zeros((ex.shape[0], 127), F32)], axis=1)
        for j in range(nch):
            pouts[j][...] = ex * t_ref[..., j * 128:(j + 1) * 128]

    return pl.pallas_call(
        body, grid=(e // eb,),
        in_specs=[
            pl.BlockSpec((eb, 128), lambda i: (i, stripe)),
            pl.BlockSpec((eb, 128), lambda i: (i, 0)),
            pl.BlockSpec((1, 16), lambda i: (0, 0)),
            pl.BlockSpec((eb, m), lambda i: (i, 0)),
        ],
        out_specs=[pl.BlockSpec((eb, 128), lambda i: (i, 0))] * (1 + nch),
        out_shape=[jax.ShapeDtypeStruct((e, 128), F32)] * (1 + nch))(
            tg, b, mx, tg)


def _tc_readout(raw, den, pw, pb, awg, awh, ab, gwi, gwh, gbi, gbh, d):
    n = raw.shape[0]

    def body(r_ref, d_ref, pw_ref, pb_ref, ag_ref, ah_ref, ab_ref,
             wi_ref, wh_ref, bi_ref, bh_ref, out_ref):
        h = r_ref[..., 0:d] / (d_ref[:, 0:1] + EPS)
        hv = jnp.dot(h, pw_ref[...], preferred_element_type=F32) + pb_ref[...]
        lg_h = jnp.dot(h, ah_ref[...], preferred_element_type=F32)
        g = jnp.sum(h, axis=0, keepdims=True)
        for _ in range(3):
            gl = jnp.dot(g, ag_ref[...], preferred_element_type=F32) + ab_ref[...]
            logits = lg_h + gl
            logits = jnp.maximum(logits, 0.01 * logits)
            mxl = jnp.max(logits, axis=0, keepdims=True)
            aa = jnp.exp(logits - mxl)
            aa = aa / jnp.sum(aa, axis=0, keepdims=True)
            context = jnp.sum(aa * hv, axis=0, keepdims=True)
            context = jnp.where(context > 0, context,
                                jnp.exp(jnp.minimum(context, 0.0)) - 1.0)
            gi = jnp.dot(context, wi_ref[...], preferred_element_type=F32) + bi_ref[...]
            gh = jnp.dot(g, wh_ref[...], preferred_element_type=F32) + bh_ref[...]
            rr = jax.nn.sigmoid(gi[:, 0:d] + gh[:, 0:d])
            zz = jax.nn.sigmoid(gi[:, d:2 * d] + gh[:, d:2 * d])
            nn_ = jnp.tanh(gi[:, 2 * d:3 * d] + rr * gh[:, 2 * d:3 * d])
            g = (1.0 - zz) * nn_ + zz * g
        out_ref[...] = g

    return pl.pallas_call(
        body,
        out_shape=jax.ShapeDtypeStruct((1, d), F32))(
            raw, den, pw, pb.reshape(1, d), awg, awh, ab.reshape(1, 1),
            gwi, gwh, gbi.reshape(1, 3 * d), gbh.reshape(1, 3 * d))


def kernel(x, edge_index, edge_attr, params):
    p = params
    n, d_in = x.shape
    e = edge_index.shape[1]
    nhalf = n // 2
    heads, hid = p['gat1_al'].shape
    d_out = p['gat3_al'].shape[1]
    src = edge_index[0]
    dst = edge_index[1]

    dsh, dsq = _tc_dstsel(dst.reshape(1, e), n)
    dsh = dsh.reshape(2 * e)
    dsq = dsq.reshape(4 * e)

    h = _tc_linear(x, p['proj_W'], p['proj_b'], act="relu", out_w=128)
    u = _tc_linear(edge_attr, p['enet_W1'], p['enet_b1'], act="relu", nb=2048)
    kk = u.shape[1]
    w2r = p['enet_W2'].reshape(kk, d_in, d_in)
    w2r = jnp.concatenate(
        [w2r, jnp.zeros((kk, 128 - d_in, d_in), F32)], axis=1)
    b2r = _pad_rows(p['enet_b2'].reshape(d_in, d_in), 128)
    gru_wi = _pad_rows(p['gru_Wi'], 128)
    gru_wh = _pad_rows(p['gru_Wh'], 128)
    hidden = h
    for _ in range(3):
        (hs,) = _sc_gather([h], [src])
        msg = _mp_messages(u, hs, w2r, b2r)
        (agg,) = _sc_scatter([msg], dsh, n)
        h = _tc_gru(agg, hidden, gru_wi, gru_wh, p['gru_bi'], p['gru_bh'],
                    d_in)
        hidden = h

    hcur = h
    for nm, dh, pre, wpad in (("gat1", hid, "none", 128),
                              ("gat2", hid, "elu", 0),
                              ("gat3", d_out, "elu", 0)):
        w = _pad_rows(p[nm + '_W'], wpad) if wpad else p[nm + '_W']
        zt, tabr, mx = _tc_gat_z(hcur, w, p[nm + '_al'], p[nm + '_ar'],
                                 heads, dh, pre=pre)
        zg, b_g = _sc_gather([zt, tabr], [src, dst])
        ex = _tc_gat_ex(zg, b_g, mx, heads, heads * dh)
        (den,) = _sc_scatter([ex], dsh, n)
        (bg,) = _sc_gather([den], [dst])
        pays = _tc_gat_pay(zg, ex, bg, heads, dh)
        outs = _sc_scatter(list(pays), dsh, n)
        hcur = outs[0] if len(outs) == 1 else jnp.concatenate(outs, axis=1)

    raw, den_prev = hcur, None
    for nm, wpad in (("ga1", 128), ("ga2", 0), ("ga3", 0)):
        aw = p[nm + '_aw']
        m = aw.shape[0] // 2
        w = _pad_rows(p[nm + '_W'], wpad) if wpad else p[nm + '_W']
        t, tabr, mx = _tc_ga_t(raw, w, p[nm + '_b'], aw[:m], aw[m:],
                               p[nm + '_ab'][0], den=den_prev)
        tg, b_g = _sc_gather([t, tabr], [src, dst])
        expays = _tc_ga_expay(tg, b_g, mx, max(p[nm + '_W'].shape[1], 128))
        (den_prev,) = _sc_scatter([expays[0]], dsh, n)
        souts = _sc_scatter(list(expays[1:]), dsh, n)
        raw = (souts[0] if len(souts) == 1
               else jnp.concatenate(souts, axis=1))

    attw = p['ro_attW']
    return _tc_readout(raw, den_prev, p['ro_pW'], p['ro_pb'],
                       attw[:d_out], attw[d_out:], p['ro_attb'][0],
                       p['ro_gWi'], p['ro_gWh'], p['ro_gbi'], p['ro_gbh'],
                       d_out)


# speedup vs baseline: 7.4390x; 1.0122x over previous
import jax
import jax.numpy as jnp
from jax import lax
from jax.experimental import pallas as pl
from jax.experimental.pallas import tpu as pltpu
from jax.experimental.pallas import tpu_sc as plsc

F32 = jnp.float32
NC, NS = 2, 16
NW = NC * NS
BIG = 1e9
EPS = 1e-16


def _sc_mesh():
    return plsc.VectorSubcoreMesh(core_axis_name="c", subcore_axis_name="s")


def _pad_rows(w, rows):
    return jnp.concatenate(
        [w, jnp.zeros((rows - w.shape[0],) + w.shape[1:], F32)], axis=0)


def _sc_gather(tables, idxs):
    e = idxs[0].shape[0]
    ns = len(tables)
    epw = e // NW
    widths = [int(t.shape[1]) for t in tables]
    chs = [64 if w <= 256 else 32 for w in widths]
    out_type = tuple(jax.ShapeDtypeStruct((e, w), F32) for w in widths)
    scratch = []
    for w, ch in zip(widths, chs):
        scratch.append(pltpu.VMEM((2, ch), jnp.int32))
        scratch.append(pltpu.VMEM((2, ch, w), F32))
        scratch.append(pltpu.SemaphoreType.DMA)
        scratch.append(pltpu.SemaphoreType.DMA)

    def body(*refs):
        tabs = refs[:ns]
        idr = refs[ns:2 * ns]
        outs = refs[2 * ns:3 * ns]
        scr = refs[3 * ns:]
        wid = lax.axis_index("s") * NC + lax.axis_index("c")
        base = wid * epw
        for i in range(ns):
            iv, rv = scr[4 * i], scr[4 * i + 1]
            sems = (scr[4 * i + 2], scr[4 * i + 3])
            ch = chs[i]
            nch = epw // ch
            handles = [None, None]
            pltpu.sync_copy(idr[i].at[pl.ds(pl.multiple_of(base, 8), ch)],
                            iv.at[0])
            handles[0] = pltpu.async_copy(tabs[i].at[iv.at[0]], rv.at[0],
                                          sems[0])
            for k in range(nch):
                cur = k & 1
                if k + 1 < nch:
                    nxt = 1 - cur
                    off1 = pl.multiple_of(base + (k + 1) * ch, 8)
                    pltpu.sync_copy(idr[i].at[pl.ds(off1, ch)], iv.at[nxt])
                    handles[nxt] = pltpu.async_copy(
                        tabs[i].at[iv.at[nxt]], rv.at[nxt], sems[nxt])
                handles[cur].wait()
                off = pl.multiple_of(base + k * ch, 8)
                pltpu.sync_copy(rv.at[cur], outs[i].at[pl.ds(off, ch)])

    return pl.kernel(body, out_type=out_type, mesh=_sc_mesh(),
                     scratch_types=tuple(scratch))(*tables, *idxs)


def _sc_scatter(payloads, dst_sel_flat, n, parts=2):
    e = payloads[0].shape[0]
    ns = len(payloads)
    phases = parts // NC
    nq = n // parts
    acc_rows = nq + 128
    stripe = acc_rows // NS
    orows = nq // NS
    ept = e // NS
    ch = 64
    widths = [int(p.shape[1]) for p in payloads]
    out_type = tuple(jax.ShapeDtypeStruct((n, w), F32) for w in widths)
    scratch = [pltpu.VMEM((2, ch), jnp.int32)]
    for w in widths:
        scratch.append(pltpu.VMEM((8, w), F32))
        scratch.append(pltpu.VMEM((2, ch, w), F32))
        scratch.append(pltpu.VMEM_SHARED((acc_rows, w), F32))
        scratch.append(pltpu.SemaphoreType.DMA)
        scratch.append(pltpu.SemaphoreType.DMA)

    def body(dsr, *refs):
        pays = refs[:ns]
        outs = refs[ns:2 * ns]
        scr = refs[2 * ns:]
        iv = scr[0]
        c = lax.axis_index("c")
        s = lax.axis_index("s")
        nch = ept // ch
        for q in range(phases):
            for i in range(ns):
                zb, acc = scr[1 + 5 * i], scr[3 + 5 * i]
                if q == 0:
                    for r in range(8):
                        for j in range(widths[i] // 16):
                            zb[r, pl.ds(j * 16, 16)] = jnp.zeros((16,), F32)
                for j in range(stripe // 8):
                    zo = pl.multiple_of(s * stripe + j * 8, 8)
                    pltpu.sync_copy(zb, acc.at[pl.ds(zo, 8)])
            plsc.subcore_barrier()
            handles = [[None, None] for _ in range(ns)]
            for k in range(nch):
                cur = k & 1
                ioff = pl.multiple_of(
                    (phases * c + q) * e + s * ept + k * ch, 8)
                eoff = pl.multiple_of(s * ept + k * ch, 8)
                for i in range(ns):
                    if handles[i][cur] is not None:
                        handles[i][cur].wait()
                pltpu.sync_copy(dsr.at[pl.ds(ioff, ch)], iv.at[cur])
                for i in range(ns):
                    pv, acc = scr[2 + 5 * i], scr[3 + 5 * i]
                    sem = scr[4 + 5 * i + cur]
                    pltpu.sync_copy(pays[i].at[pl.ds(eoff, ch)], pv.at[cur])
                    handles[i][cur] = pltpu.async_copy(
                        pv.at[cur], acc.at[iv.at[cur]], sem, add=True)
            for i in range(ns):
                for par in range(2):
                    if handles[i][par] is not None:
                        handles[i][par].wait()
            plsc.subcore_barrier()
            so = pl.multiple_of(s * orows, 8)
            oo = pl.multiple_of((phases * c + q) * nq + s * orows, 8)
            for i in range(ns):
                acc = scr[3 + 5 * i]
                pltpu.sync_copy(acc.at[pl.ds(so, orows)],
                                outs[i].at[pl.ds(oo, orows)])
            plsc.subcore_barrier()

    return pl.kernel(body, out_type=out_type, mesh=_sc_mesh(),
                     scratch_types=tuple(scratch))(dst_sel_flat, *payloads)


def _tc_dstsel(dst2d, n):
    e = dst2d.shape[1]
    eb = 2048

    def body(d_ref, o_ref):
        d = d_ref[...]
        r = n // 2
        sels = []
        for q in range(2):
            dq = d - q * r
            sels.append(jnp.where((dq >= 0) & (dq < r), dq, r))
        o_ref[...] = jnp.concatenate(sels, axis=0)

    return pl.pallas_call(
        body, grid=(e // eb,),
        in_specs=[pl.BlockSpec((1, eb), lambda i: (0, i))],
        out_specs=pl.BlockSpec((2, eb), lambda i: (0, i)),
        out_shape=jax.ShapeDtypeStruct((2, e), jnp.int32))(dst2d)


def _pre_act(xb, pre, d=None):
    if d is not None:
        xb = xb / (d[:, 0:1] + EPS)
    if pre == "relu":
        xb = jnp.maximum(xb, 0.0)
    elif pre == "elu":
        xb = jnp.where(xb > 0, xb, jnp.exp(jnp.minimum(xb, 0.0)) - 1.0)
    return xb


def _tc_linear(x, w, b, act="none", out_w=None, nb=1024):
    n, kdim = x.shape
    m = w.shape[-1]
    mo = m if out_w is None else out_w

    def body(x_ref, w_ref, b_ref, out_ref):
        y = jnp.dot(x_ref[...], w_ref[...], preferred_element_type=F32)
        y = y + b_ref[...]
        if act == "relu":
            y = jnp.maximum(y, 0.0)
        if mo > m:
            y = jnp.concatenate(
                [y, jnp.zeros((y.shape[0], mo - m), F32)], axis=1)
        out_ref[...] = y

    return pl.pallas_call(
        body, grid=(n // nb,),
        in_specs=[pl.BlockSpec((nb, kdim), lambda i: (i, 0)),
                  pl.BlockSpec((kdim, m), lambda i: (0, 0)),
                  pl.BlockSpec((1, m), lambda i: (0, 0))],
        out_specs=pl.BlockSpec((nb, mo), lambda i: (i, 0)),
        out_shape=jax.ShapeDtypeStruct((n, mo), F32))(x, w, b.reshape(1, m))


def _mp_msg_body(u_ref, hs_ref, w2r_ref, b2r_ref, out_ref):
    u = u_ref[...]
    hs = hs_ref[...]
    d = b2r_ref.shape[1]
    acc = jnp.dot(hs, b2r_ref[...], preferred_element_type=F32)
    for k in range(u.shape[1]):
        acc += u[:, k:k + 1] * jnp.dot(hs, w2r_ref[k],
                                       preferred_element_type=F32)
    acc = jnp.concatenate([acc, jnp.zeros((acc.shape[0], 128 - d), F32)],
                          axis=1)
    out_ref[...] = acc


def _mp_messages(u, hs, w2r, b2r):
    e = hs.shape[0]
    d = b2r.shape[1]
    kk = u.shape[1]
    eb = 4096
    return pl.pallas_call(
        _mp_msg_body,
        grid=(e // eb,),
        in_specs=[
            pl.BlockSpec((eb, kk), lambda i: (i, 0)),
            pl.BlockSpec((eb, 128), lambda i: (i, 0)),
            pl.BlockSpec((kk, 128, d), lambda i: (0, 0, 0)),
            pl.BlockSpec((128, d), lambda i: (0, 0)),
        ],
        out_specs=pl.BlockSpec((eb, 128), lambda i: (i, 0)),
        out_shape=jax.ShapeDtypeStruct((e, 128), F32))(u, hs, w2r, b2r)


def _tc_gru(agg, hidden, wi, wh, bi, bh, d, nb=2048):
    n = hidden.shape[0]

    def body(a_ref, h_ref, wi_ref, wh_ref, bi_ref, bh_ref, out_ref):
        m = jnp.maximum(a_ref[...], 0.0)
        hp = h_ref[...]
        gi = jnp.dot(m, wi_ref[...], preferred_element_type=F32) + bi_ref[...]
        gh = jnp.dot(hp, wh_ref[...], preferred_element_type=F32) + bh_ref[...]
        r = jax.nn.sigmoid(gi[:, 0:d] + gh[:, 0:d])
        z = jax.nn.sigmoid(gi[:, d:2 * d] + gh[:, d:2 * d])
        nn_ = jnp.tanh(gi[:, 2 * d:3 * d] + r * gh[:, 2 * d:3 * d])
        y = (1.0 - z) * nn_ + z * hp[:, 0:d]
        out_ref[...] = jnp.concatenate(
            [y, jnp.zeros((y.shape[0], 128 - d), F32)], axis=1)

    return pl.pallas_call(
        body, grid=(n // nb,),
        in_specs=[
            pl.BlockSpec((nb, 128), lambda i: (i, 0)),
            pl.BlockSpec((nb, 128), lambda i: (i, 0)),
            pl.BlockSpec((128, 3 * d), lambda i: (0, 0)),
            pl.BlockSpec((128, 3 * d), lambda i: (0, 0)),
            pl.BlockSpec((1, 3 * d), lambda i: (0, 0)),
            pl.BlockSpec((1, 3 * d), lambda i: (0, 0)),
        ],
        out_specs=pl.BlockSpec((nb, 128), lambda i: (i, 0)),
        out_shape=jax.ShapeDtypeStruct((n, 128), F32))(
            agg, hidden, wi, wh, bi.reshape(1, 3 * d), bh.reshape(1, 3 * d))


def _tc_gat_z(x, w, al, ar, heads, dh, pre="none", nb=1024):
    n, kdim = x.shape
    m = heads * dh

    def body(x_ref, w_ref, al_ref, ar_ref, zt_ref, tab_ref, mx_ref):
        i = pl.program_id(0)
        xb = _pre_act(x_ref[...], pre)
        z = jnp.dot(xb, w_ref[...], preferred_element_type=F32)
        els, ers = [], []
        for h in range(heads):
            zh = z[:, h * dh:(h + 1) * dh]
            els.append(jnp.sum(zh * al_ref[h:h + 1, :], axis=1, keepdims=True))
            ers.append(jnp.sum(zh * ar_ref[h:h + 1, :], axis=1, keepdims=True))
        el = jnp.concatenate(els, axis=1)
        er = jnp.concatenate(ers, axis=1)
        pad = jnp.zeros((xb.shape[0], 128 - heads), F32)
        zt_ref[...] = jnp.concatenate([z, el, pad], axis=1)
        tab_ref[...] = jnp.concatenate([er, pad], axis=1)
        cur = jnp.concatenate(
            [jnp.max(el, axis=0, keepdims=True),
             jnp.max(er, axis=0, keepdims=True),
             jnp.full((1, 16 - 2 * heads), -BIG, F32)], axis=1)

        @pl.when(i == 0)
        def _():
            mx_ref[...] = cur

        @pl.when(i > 0)
        def _():
            mx_ref[...] = jnp.maximum(mx_ref[...], cur)

    return pl.pallas_call(
        body, grid=(n // nb,),
        in_specs=[
            pl.BlockSpec((nb, kdim), lambda i: (i, 0)),
            pl.BlockSpec((kdim, m), lambda i: (0, 0)),
            pl.BlockSpec((heads, dh), lambda i: (0, 0)),
            pl.BlockSpec((heads, dh), lambda i: (0, 0)),
        ],
        out_specs=[
            pl.BlockSpec((nb, m + 128), lambda i: (i, 0)),
            pl.BlockSpec((nb, 128), lambda i: (i, 0)),
            pl.BlockSpec((1, 16), lambda i: (0, 0)),
        ],
        out_shape=[
            jax.ShapeDtypeStruct((n, m + 128), F32),
            jax.ShapeDtypeStruct((n, 128), F32),
            jax.ShapeDtypeStruct((1, 16), F32),
        ])(x, w, al, ar)


def _tc_gat_ex(zg, b, mx, heads, m, eb=4096):
    e = b.shape[0]
    stripe = m // 128

    def body(a_ref, b_ref, mx_ref, out_ref):
        mm = mx_ref[...]
        s = mm[:, 0:heads] + mm[:, heads:2 * heads]
        s = jnp.maximum(s, 0.2 * s)
        ee = a_ref[..., 0:heads] + b_ref[..., 0:heads]
        ee = jnp.maximum(ee, 0.2 * ee)
        ex = jnp.exp(ee - s)
        out_ref[...] = jnp.concatenate(
            [ex, jnp.zeros((ex.shape[0], 128 - heads), F32)], axis=1)

    return pl.pallas_call(
        body, grid=(e // eb,),
        in_specs=[
            pl.BlockSpec((eb, 128), lambda i: (i, stripe)),
            pl.BlockSpec((eb, 128), lambda i: (i, 0)),
            pl.BlockSpec((1, 16), lambda i: (0, 0)),
        ],
        out_specs=pl.BlockSpec((eb, 128), lambda i: (i, 0)),
        out_shape=jax.ShapeDtypeStruct((e, 128), F32))(zg, b, mx)


def _tc_binv(den, nb=1024):
    n = den.shape[0]

    def body(d_ref, o_ref):
        o_ref[...] = 1.0 / (d_ref[...] + EPS)

    return pl.pallas_call(
        body, grid=(n // nb,),
        in_specs=[pl.BlockSpec((nb, 128), lambda i: (i, 0))],
        out_specs=pl.BlockSpec((nb, 128), lambda i: (i, 0)),
        out_shape=jax.ShapeDtypeStruct((n, 128), F32))(den)


def _tc_gat_pay(zg, ex, bg, heads, dh, eb=2048):
    e = zg.shape[0]
    m = heads * dh
    mo = max(dh, 128)
    nch = mo // 128

    def body(z_ref, e_ref, b_ref, *outs):
        w = (e_ref[..., 0:heads] / (b_ref[..., 0:heads] + EPS)
             * (1.0 / heads))
        z = z_ref[...]
        acc = w[:, 0:1] * z[:, 0:dh]
        for h in range(1, heads):
            acc += w[:, h:h + 1] * z[:, h * dh:(h + 1) * dh]
        if mo > dh:
            acc = jnp.concatenate(
                [acc, jnp.zeros((acc.shape[0], mo - dh), F32)], axis=1)
        for j in range(nch):
            outs[j][...] = acc[:, j * 128:(j + 1) * 128]

    return pl.pallas_call(
        body, grid=(e // eb,),
        in_specs=[
            pl.BlockSpec((eb, m), lambda i: (i, 0)),
            pl.BlockSpec((eb, 128), lambda i: (i, 0)),
            pl.BlockSpec((eb, 128), lambda i: (i, 0)),
        ],
        out_specs=[pl.BlockSpec((eb, 128), lambda i: (i, 0))] * nch,
        out_shape=[jax.ShapeDtypeStruct((e, 128), F32)] * nch)(zg, ex, bg)


def _tc_ga_t(x, w, b, awl, awr, ab, pre="none", den=None, nb=1024):
    n, kdim = x.shape
    m = w.shape[1]
    mo = max(m, 128)
    ins = [x, w, b.reshape(1, m), awl, awr, ab.reshape(1, 1)]
    specs = [pl.BlockSpec((nb, kdim), lambda i: (i, 0)),
             pl.BlockSpec((kdim, m), lambda i: (0, 0)),
             pl.BlockSpec((1, m), lambda i: (0, 0)),
             pl.BlockSpec((m, 1), lambda i: (0, 0)),
             pl.BlockSpec((m, 1), lambda i: (0, 0)),
             pl.BlockSpec((1, 1), lambda i: (0, 0))]
    if den is not None:
        ins.append(den)
        specs.append(pl.BlockSpec((nb, 128), lambda i: (i, 0)))

    def body(x_ref, w_ref, b_ref, awl_ref, awr_ref, ab_ref, *rest):
        t_ref, tab_ref, mx_ref = rest[-3:]
        d = rest[0][...] if den is not None else None
        i = pl.program_id(0)
        xb = _pre_act(x_ref[...], pre, d)
        t = jnp.dot(xb, w_ref[...], preferred_element_type=F32) + b_ref[...]
        tl = jnp.dot(t, awl_ref[...], preferred_element_type=F32) + ab_ref[...]
        tr = jnp.dot(t, awr_ref[...], preferred_element_type=F32)
        if mo > m:
            t = jnp.concatenate(
                [t, jnp.zeros((t.shape[0], mo - m), F32)], axis=1)
        pad = jnp.zeros((xb.shape[0], 127), F32)
        t_ref[...] = jnp.concatenate([t, tl, pad], axis=1)
        tab_ref[...] = jnp.concatenate([tr, pad], axis=1)
        cur = jnp.concatenate(
            [jnp.max(tl, axis=0, keepdims=True),
             jnp.max(tr, axis=0, keepdims=True),
             jnp.full((1, 14), -BIG, F32)], axis=1)

        @pl.when(i == 0)
        def _():
            mx_ref[...] = cur

        @pl.when(i > 0)
        def _():
            mx_ref[...] = jnp.maximum(mx_ref[...], cur)

    return pl.pallas_call(
        body, grid=(n // nb,), in_specs=specs,
        out_specs=[
            pl.BlockSpec((nb, mo + 128), lambda i: (i, 0)),
            pl.BlockSpec((nb, 128), lambda i: (i, 0)),
            pl.BlockSpec((1, 16), lambda i: (0, 0)),
        ],
        out_shape=[
            jax.ShapeDtypeStruct((n, mo + 128), F32),
            jax.ShapeDtypeStruct((n, 128), F32),
            jax.ShapeDtypeStruct((1, 16), F32),
        ])(*ins)


def _tc_ga_expay(tg, b, mx, m, eb=4096):
    e = b.shape[0]
    nch = m // 128
    stripe = nch

    def body(a_ref, b_ref, mx_ref, t_ref, ex_ref, *pouts):
        mxv = mx_ref[...]
        s = mxv[:, 0:1] + mxv[:, 1:2]
        s = jnp.maximum(s, 0.2 * s)
        ee = a_ref[..., 0:1] + b_ref[..., 0:1]
        ee = jnp.maximum(ee, 0.2 * ee)
        ex = jnp.exp(ee - s)
        ex_ref[...] = jnp.concatenate(
            [ex, jnp.zeros((ex.shape[0], 127), F32)], axis=1)
        for j in range(nch):
            pouts[j][...] = ex * t_ref[..., j * 128:(j + 1) * 128]

    return pl.pallas_call(
        body, grid=(e // eb,),
        in_specs=[
            pl.BlockSpec((eb, 128), lambda i: (i, stripe)),
            pl.BlockSpec((eb, 128), lambda i: (i, 0)),
            pl.BlockSpec((1, 16), lambda i: (0, 0)),
            pl.BlockSpec((eb, m), lambda i: (i, 0)),
        ],
        out_specs=[pl.BlockSpec((eb, 128), lambda i: (i, 0))] * (1 + nch),
        out_shape=[jax.ShapeDtypeStruct((e, 128), F32)] * (1 + nch))(
            tg, b, mx, tg)


def _tc_readout(raw, den, pw, pb, awg, awh, ab, gwi, gwh, gbi, gbh, d):
    n = raw.shape[0]

    def body(r_ref, d_ref, pw_ref, pb_ref, ag_ref, ah_ref, ab_ref,
             wi_ref, wh_ref, bi_ref, bh_ref, out_ref):
        h = r_ref[..., 0:d] / (d_ref[:, 0:1] + EPS)
        hv = jnp.dot(h, pw_ref[...], preferred_element_type=F32) + pb_ref[...]
        lg_h = jnp.dot(h, ah_ref[...], preferred_element_type=F32)
        g = jnp.sum(h, axis=0, keepdims=True)
        for _ in range(3):
            gl = jnp.dot(g, ag_ref[...], preferred_element_type=F32) + ab_ref[...]
            logits = lg_h + gl
            logits = jnp.maximum(logits, 0.01 * logits)
            mxl = jnp.max(logits, axis=0, keepdims=True)
            aa = jnp.exp(logits - mxl)
            aa = aa / jnp.sum(aa, axis=0, keepdims=True)
            context = jnp.sum(aa * hv, axis=0, keepdims=True)
            context = jnp.where(context > 0, context,
                                jnp.exp(jnp.minimum(context, 0.0)) - 1.0)
            gi = jnp.dot(context, wi_ref[...], preferred_element_type=F32) + bi_ref[...]
            gh = jnp.dot(g, wh_ref[...], preferred_element_type=F32) + bh_ref[...]
            rr = jax.nn.sigmoid(gi[:, 0:d] + gh[:, 0:d])
            zz = jax.nn.sigmoid(gi[:, d:2 * d] + gh[:, d:2 * d])
            nn_ = jnp.tanh(gi[:, 2 * d:3 * d] + rr * gh[:, 2 * d:3 * d])
            g = (1.0 - zz) * nn_ + zz * g
        out_ref[...] = g

    return pl.pallas_call(
        body,
        out_shape=jax.ShapeDtypeStruct((1, d), F32))(
            raw, den, pw, pb.reshape(1, d), awg, awh, ab.reshape(1, 1),
            gwi, gwh, gbi.reshape(1, 3 * d), gbh.reshape(1, 3 * d))


def kernel(x, edge_index, edge_attr, params):
    p = params
    n, d_in = x.shape
    e = edge_index.shape[1]
    nhalf = n // 2
    heads, hid = p['gat1_al'].shape
    d_out = p['gat3_al'].shape[1]
    src = edge_index[0]
    dst = edge_index[1]

    dsh = _tc_dstsel(dst.reshape(1, e), n).reshape(2 * e)

    h = _tc_linear(x, p['proj_W'], p['proj_b'], act="relu", out_w=128)
    u = _tc_linear(edge_attr, p['enet_W1'], p['enet_b1'], act="relu", nb=2048)
    kk = u.shape[1]
    w2r = p['enet_W2'].reshape(kk, d_in, d_in)
    w2r = jnp.concatenate(
        [w2r, jnp.zeros((kk, 128 - d_in, d_in), F32)], axis=1)
    b2r = _pad_rows(p['enet_b2'].reshape(d_in, d_in), 128)
    gru_wi = _pad_rows(p['gru_Wi'], 128)
    gru_wh = _pad_rows(p['gru_Wh'], 128)
    hidden = h
    for _ in range(3):
        (hs,) = _sc_gather([h], [src])
        msg = _mp_messages(u, hs, w2r, b2r)
        (agg,) = _sc_scatter([msg], dsh, n)
        h = _tc_gru(agg, hidden, gru_wi, gru_wh, p['gru_bi'], p['gru_bh'],
                    d_in)
        hidden = h

    hcur = h
    for nm, dh, pre, wpad in (("gat1", hid, "none", 128),
                              ("gat2", hid, "elu", 0),
                              ("gat3", d_out, "elu", 0)):
        w = _pad_rows(p[nm + '_W'], wpad) if wpad else p[nm + '_W']
        zt, tabr, mx = _tc_gat_z(hcur, w, p[nm + '_al'], p[nm + '_ar'],
                                 heads, dh, pre=pre)
        zg, b_g = _sc_gather([zt, tabr], [src, dst])
        ex = _tc_gat_ex(zg, b_g, mx, heads, heads * dh)
        (den,) = _sc_scatter([ex], dsh, n)
        (bg,) = _sc_gather([den], [dst])
        pays = _tc_gat_pay(zg, ex, bg, heads, dh)
        outs = _sc_scatter(list(pays), dsh, n)
        hcur = outs[0] if len(outs) == 1 else jnp.concatenate(outs, axis=1)

    raw, den_prev = hcur, None
    for nm, wpad in (("ga1", 128), ("ga2", 0), ("ga3", 0)):
        aw = p[nm + '_aw']
        m = aw.shape[0] // 2
        w = _pad_rows(p[nm + '_W'], wpad) if wpad else p[nm + '_W']
        t, tabr, mx = _tc_ga_t(raw, w, p[nm + '_b'], aw[:m], aw[m:],
                               p[nm + '_ab'][0], den=den_prev)
        tg, b_g = _sc_gather([t, tabr], [src, dst])
        expays = _tc_ga_expay(tg, b_g, mx, max(p[nm + '_W'].shape[1], 128))
        (den_prev,) = _sc_scatter([expays[0]], dsh, n)
        souts = _sc_scatter(list(expays[1:]), dsh, n)
        raw = (souts[0] if len(souts) == 1
               else jnp.concatenate(souts, axis=1))

    attw = p['ro_attW']
    return _tc_readout(raw, den_prev, p['ro_pW'], p['ro_pb'],
                       attw[:d_out], attw[d_out:], p['ro_attb'][0],
                       p['ro_gWi'], p['ro_gWh'], p['ro_gbi'], p['ro_gbh'],
                       d_out)


# speedup vs baseline: 8.0996x; 1.0888x over previous
import jax
import jax.numpy as jnp
from jax import lax
from jax.experimental import pallas as pl
from jax.experimental.pallas import tpu as pltpu
from jax.experimental.pallas import tpu_sc as plsc

F32 = jnp.float32
NC, NS = 2, 16
NW = NC * NS
BIG = 1e9
EPS = 1e-16


def _sc_mesh():
    return plsc.VectorSubcoreMesh(core_axis_name="c", subcore_axis_name="s")


def _pad_rows(w, rows):
    return jnp.concatenate(
        [w, jnp.zeros((rows - w.shape[0],) + w.shape[1:], F32)], axis=0)


def _sc_gather(tables, idxs):
    e = idxs[0].shape[0]
    ns = len(tables)
    epw = e // NW
    widths = [int(t.shape[1]) for t in tables]
    chs = [128 if w <= 128 else (64 if w <= 512 else 32)
           for w in widths]
    out_type = tuple(jax.ShapeDtypeStruct((e, w), F32) for w in widths)
    scratch = []
    for w, ch in zip(widths, chs):
        scratch.append(pltpu.VMEM((2, ch), jnp.int32))
        scratch.append(pltpu.VMEM((2, ch, w), F32))
        scratch.append(pltpu.SemaphoreType.DMA)
        scratch.append(pltpu.SemaphoreType.DMA)

    def body(*refs):
        tabs = refs[:ns]
        idr = refs[ns:2 * ns]
        outs = refs[2 * ns:3 * ns]
        scr = refs[3 * ns:]
        wid = lax.axis_index("s") * NC + lax.axis_index("c")
        base = wid * epw
        for i in range(ns):
            iv, rv = scr[4 * i], scr[4 * i + 1]
            sems = (scr[4 * i + 2], scr[4 * i + 3])
            ch = chs[i]
            nch = epw // ch
            handles = [None, None]
            pltpu.sync_copy(idr[i].at[pl.ds(pl.multiple_of(base, 8), ch)],
                            iv.at[0])
            handles[0] = pltpu.async_copy(tabs[i].at[iv.at[0]], rv.at[0],
                                          sems[0])
            for k in range(nch):
                cur = k & 1
                if k + 1 < nch:
                    nxt = 1 - cur
                    off1 = pl.multiple_of(base + (k + 1) * ch, 8)
                    pltpu.sync_copy(idr[i].at[pl.ds(off1, ch)], iv.at[nxt])
                    handles[nxt] = pltpu.async_copy(
                        tabs[i].at[iv.at[nxt]], rv.at[nxt], sems[nxt])
                handles[cur].wait()
                off = pl.multiple_of(base + k * ch, 8)
                pltpu.sync_copy(rv.at[cur], outs[i].at[pl.ds(off, ch)])

    return pl.kernel(body, out_type=out_type, mesh=_sc_mesh(),
                     scratch_types=tuple(scratch))(*tables, *idxs)


def _sc_scatter(payloads, dst_sel_flat, n, parts=2):
    e = payloads[0].shape[0]
    ns = len(payloads)
    phases = parts // NC
    nq = n // parts
    acc_rows = nq + 128
    stripe = acc_rows // NS
    orows = nq // NS
    ept = e // NS
    widths = [int(p.shape[1]) for p in payloads]
    ch = 128 if len(payloads) == 1 else 64
    out_type = tuple(jax.ShapeDtypeStruct((n, w), F32) for w in widths)
    scratch = [pltpu.VMEM((2, ch), jnp.int32)]
    for w in widths:
        scratch.append(pltpu.VMEM((8, w), F32))
        scratch.append(pltpu.VMEM((2, ch, w), F32))
        scratch.append(pltpu.VMEM_SHARED((acc_rows, w), F32))
        scratch.append(pltpu.SemaphoreType.DMA)
        scratch.append(pltpu.SemaphoreType.DMA)

    def body(dsr, *refs):
        pays = refs[:ns]
        outs = refs[ns:2 * ns]
        scr = refs[2 * ns:]
        iv = scr[0]
        c = lax.axis_index("c")
        s = lax.axis_index("s")
        nch = ept // ch
        for q in range(phases):
            for i in range(ns):
                zb, acc = scr[1 + 5 * i], scr[3 + 5 * i]
                if q == 0:
                    for r in range(8):
                        for j in range(widths[i] // 16):
                            zb[r, pl.ds(j * 16, 16)] = jnp.zeros((16,), F32)
                for j in range(stripe // 8):
                    zo = pl.multiple_of(s * stripe + j * 8, 8)
                    pltpu.sync_copy(zb, acc.at[pl.ds(zo, 8)])
            plsc.subcore_barrier()
            handles = [[None, None] for _ in range(ns)]
            for k in range(nch):
                cur = k & 1
                ioff = pl.multiple_of(
                    (phases * c + q) * e + s * ept + k * ch, 8)
                eoff = pl.multiple_of(s * ept + k * ch, 8)
                for i in range(ns):
                    if handles[i][cur] is not None:
                        handles[i][cur].wait()
                pltpu.sync_copy(dsr.at[pl.ds(ioff, ch)], iv.at[cur])
                for i in range(ns):
                    pv, acc = scr[2 + 5 * i], scr[3 + 5 * i]
                    sem = scr[4 + 5 * i + cur]
                    pltpu.sync_copy(pays[i].at[pl.ds(eoff, ch)], pv.at[cur])
                    handles[i][cur] = pltpu.async_copy(
                        pv.at[cur], acc.at[iv.at[cur]], sem, add=True)
            for i in range(ns):
                for par in range(2):
                    if handles[i][par] is not None:
                        handles[i][par].wait()
            plsc.subcore_barrier()
            so = pl.multiple_of(s * orows, 8)
            oo = pl.multiple_of((phases * c + q) * nq + s * orows, 8)
            for i in range(ns):
                acc = scr[3 + 5 * i]
                pltpu.sync_copy(acc.at[pl.ds(so, orows)],
                                outs[i].at[pl.ds(oo, orows)])
            plsc.subcore_barrier()

    return pl.kernel(body, out_type=out_type, mesh=_sc_mesh(),
                     scratch_types=tuple(scratch))(dst_sel_flat, *payloads)


def _tc_dstsel(dst2d, n):
    e = dst2d.shape[1]
    eb = 2048

    def body(d_ref, o_ref):
        d = d_ref[...]
        r = n // 2
        sels = []
        for q in range(2):
            dq = d - q * r
            sels.append(jnp.where((dq >= 0) & (dq < r), dq, r))
        o_ref[...] = jnp.concatenate(sels, axis=0)

    return pl.pallas_call(
        body, grid=(e // eb,),
        in_specs=[pl.BlockSpec((1, eb), lambda i: (0, i))],
        out_specs=pl.BlockSpec((2, eb), lambda i: (0, i)),
        out_shape=jax.ShapeDtypeStruct((2, e), jnp.int32))(dst2d)


def _pre_act(xb, pre, d=None):
    if d is not None:
        xb = xb / (d[:, 0:1] + EPS)
    if pre == "relu":
        xb = jnp.maximum(xb, 0.0)
    elif pre == "elu":
        xb = jnp.where(xb > 0, xb, jnp.exp(jnp.minimum(xb, 0.0)) - 1.0)
    return xb


def _tc_linear(x, w, b, act="none", out_w=None, nb=1024):
    n, kdim = x.shape
    m = w.shape[-1]
    mo = m if out_w is None else out_w

    def body(x_ref, w_ref, b_ref, out_ref):
        y = jnp.dot(x_ref[...], w_ref[...], preferred_element_type=F32)
        y = y + b_ref[...]
        if act == "relu":
            y = jnp.maximum(y, 0.0)
        if mo > m:
            y = jnp.concatenate(
                [y, jnp.zeros((y.shape[0], mo - m), F32)], axis=1)
        out_ref[...] = y

    return pl.pallas_call(
        body, grid=(n // nb,),
        in_specs=[pl.BlockSpec((nb, kdim), lambda i: (i, 0)),
                  pl.BlockSpec((kdim, m), lambda i: (0, 0)),
                  pl.BlockSpec((1, m), lambda i: (0, 0))],
        out_specs=pl.BlockSpec((nb, mo), lambda i: (i, 0)),
        out_shape=jax.ShapeDtypeStruct((n, mo), F32))(x, w, b.reshape(1, m))


def _mp_msg_body(u_ref, hs_ref, w2r_ref, b2r_ref, out_ref):
    u = u_ref[...]
    hs = hs_ref[...]
    d = b2r_ref.shape[1]
    acc = jnp.dot(hs, b2r_ref[...], preferred_element_type=F32)
    for k in range(u.shape[1]):
        acc += u[:, k:k + 1] * jnp.dot(hs, w2r_ref[k],
                                       preferred_element_type=F32)
    acc = jnp.concatenate([acc, jnp.zeros((acc.shape[0], 128 - d), F32)],
                          axis=1)
    out_ref[...] = acc


def _mp_messages(u, hs, w2r, b2r):
    e = hs.shape[0]
    d = b2r.shape[1]
    kk = u.shape[1]
    eb = 4096
    return pl.pallas_call(
        _mp_msg_body,
        grid=(e // eb,),
        in_specs=[
            pl.BlockSpec((eb, kk), lambda i: (i, 0)),
            pl.BlockSpec((eb, 128), lambda i: (i, 0)),
            pl.BlockSpec((kk, 128, d), lambda i: (0, 0, 0)),
            pl.BlockSpec((128, d), lambda i: (0, 0)),
        ],
        out_specs=pl.BlockSpec((eb, 128), lambda i: (i, 0)),
        out_shape=jax.ShapeDtypeStruct((e, 128), F32))(u, hs, w2r, b2r)


def _tc_gru(agg, hidden, wi, wh, bi, bh, d, nb=2048):
    n = hidden.shape[0]

    def body(a_ref, h_ref, wi_ref, wh_ref, bi_ref, bh_ref, out_ref):
        m = jnp.maximum(a_ref[...], 0.0)
        hp = h_ref[...]
        gi = jnp.dot(m, wi_ref[...], preferred_element_type=F32) + bi_ref[...]
        gh = jnp.dot(hp, wh_ref[...], preferred_element_type=F32) + bh_ref[...]
        r = jax.nn.sigmoid(gi[:, 0:d] + gh[:, 0:d])
        z = jax.nn.sigmoid(gi[:, d:2 * d] + gh[:, d:2 * d])
        nn_ = jnp.tanh(gi[:, 2 * d:3 * d] + r * gh[:, 2 * d:3 * d])
        y = (1.0 - z) * nn_ + z * hp[:, 0:d]
        out_ref[...] = jnp.concatenate(
            [y, jnp.zeros((y.shape[0], 128 - d), F32)], axis=1)

    return pl.pallas_call(
        body, grid=(n // nb,),
        in_specs=[
            pl.BlockSpec((nb, 128), lambda i: (i, 0)),
            pl.BlockSpec((nb, 128), lambda i: (i, 0)),
            pl.BlockSpec((128, 3 * d), lambda i: (0, 0)),
            pl.BlockSpec((128, 3 * d), lambda i: (0, 0)),
            pl.BlockSpec((1, 3 * d), lambda i: (0, 0)),
            pl.BlockSpec((1, 3 * d), lambda i: (0, 0)),
        ],
        out_specs=pl.BlockSpec((nb, 128), lambda i: (i, 0)),
        out_shape=jax.ShapeDtypeStruct((n, 128), F32))(
            agg, hidden, wi, wh, bi.reshape(1, 3 * d), bh.reshape(1, 3 * d))


def _tc_gat_z(x, w, al, ar, heads, dh, pre="none", nb=1024):
    n, kdim = x.shape
    m = heads * dh

    def body(x_ref, w_ref, al_ref, ar_ref, zt_ref, tab_ref, mx_ref):
        i = pl.program_id(0)
        xb = _pre_act(x_ref[...], pre)
        z = jnp.dot(xb, w_ref[...], preferred_element_type=F32)
        els, ers = [], []
        for h in range(heads):
            zh = z[:, h * dh:(h + 1) * dh]
            els.append(jnp.sum(zh * al_ref[h:h + 1, :], axis=1, keepdims=True))
            ers.append(jnp.sum(zh * ar_ref[h:h + 1, :], axis=1, keepdims=True))
        el = jnp.concatenate(els, axis=1)
        er = jnp.concatenate(ers, axis=1)
        pad = jnp.zeros((xb.shape[0], 128 - heads), F32)
        zt_ref[...] = jnp.concatenate([z, el, pad], axis=1)
        tab_ref[...] = jnp.concatenate([er, pad], axis=1)
        cur = jnp.concatenate(
            [jnp.max(el, axis=0, keepdims=True),
             jnp.max(er, axis=0, keepdims=True),
             jnp.full((1, 16 - 2 * heads), -BIG, F32)], axis=1)

        @pl.when(i == 0)
        def _():
            mx_ref[...] = cur

        @pl.when(i > 0)
        def _():
            mx_ref[...] = jnp.maximum(mx_ref[...], cur)

    return pl.pallas_call(
        body, grid=(n // nb,),
        in_specs=[
            pl.BlockSpec((nb, kdim), lambda i: (i, 0)),
            pl.BlockSpec((kdim, m), lambda i: (0, 0)),
            pl.BlockSpec((heads, dh), lambda i: (0, 0)),
            pl.BlockSpec((heads, dh), lambda i: (0, 0)),
        ],
        out_specs=[
            pl.BlockSpec((nb, m + 128), lambda i: (i, 0)),
            pl.BlockSpec((nb, 128), lambda i: (i, 0)),
            pl.BlockSpec((1, 16), lambda i: (0, 0)),
        ],
        out_shape=[
            jax.ShapeDtypeStruct((n, m + 128), F32),
            jax.ShapeDtypeStruct((n, 128), F32),
            jax.ShapeDtypeStruct((1, 16), F32),
        ])(x, w, al, ar)


def _tc_gat_ex(zg, b, mx, heads, m, eb=4096):
    e = b.shape[0]
    stripe = m // 128

    def body(a_ref, b_ref, mx_ref, out_ref):
        mm = mx_ref[...]
        s = mm[:, 0:heads] + mm[:, heads:2 * heads]
        s = jnp.maximum(s, 0.2 * s)
        ee = a_ref[..., 0:heads] + b_ref[..., 0:heads]
        ee = jnp.maximum(ee, 0.2 * ee)
        ex = jnp.exp(ee - s)
        out_ref[...] = jnp.concatenate(
            [ex, jnp.zeros((ex.shape[0], 128 - heads), F32)], axis=1)

    return pl.pallas_call(
        body, grid=(e // eb,),
        in_specs=[
            pl.BlockSpec((eb, 128), lambda i: (i, stripe)),
            pl.BlockSpec((eb, 128), lambda i: (i, 0)),
            pl.BlockSpec((1, 16), lambda i: (0, 0)),
        ],
        out_specs=pl.BlockSpec((eb, 128), lambda i: (i, 0)),
        out_shape=jax.ShapeDtypeStruct((e, 128), F32))(zg, b, mx)


def _tc_binv(den, nb=1024):
    n = den.shape[0]

    def body(d_ref, o_ref):
        o_ref[...] = 1.0 / (d_ref[...] + EPS)

    return pl.pallas_call(
        body, grid=(n // nb,),
        in_specs=[pl.BlockSpec((nb, 128), lambda i: (i, 0))],
        out_specs=pl.BlockSpec((nb, 128), lambda i: (i, 0)),
        out_shape=jax.ShapeDtypeStruct((n, 128), F32))(den)


def _tc_gat_pay(zg, ex, bg, heads, dh, eb=2048):
    e = zg.shape[0]
    m = heads * dh
    mo = max(dh, 128)
    nch = mo // 128

    def body(z_ref, e_ref, b_ref, *outs):
        w = (e_ref[..., 0:heads] / (b_ref[..., 0:heads] + EPS)
             * (1.0 / heads))
        z = z_ref[...]
        acc = w[:, 0:1] * z[:, 0:dh]
        for h in range(1, heads):
            acc += w[:, h:h + 1] * z[:, h * dh:(h + 1) * dh]
        if mo > dh:
            acc = jnp.concatenate(
                [acc, jnp.zeros((acc.shape[0], mo - dh), F32)], axis=1)
        for j in range(nch):
            outs[j][...] = acc[:, j * 128:(j + 1) * 128]

    return pl.pallas_call(
        body, grid=(e // eb,),
        in_specs=[
            pl.BlockSpec((eb, m), lambda i: (i, 0)),
            pl.BlockSpec((eb, 128), lambda i: (i, 0)),
            pl.BlockSpec((eb, 128), lambda i: (i, 0)),
        ],
        out_specs=[pl.BlockSpec((eb, 128), lambda i: (i, 0))] * nch,
        out_shape=[jax.ShapeDtypeStruct((e, 128), F32)] * nch)(zg, ex, bg)


def _tc_ga_t(x, w, b, awl, awr, ab, pre="none", den=None, nb=1024):
    n, kdim = x.shape
    m = w.shape[1]
    mo = max(m, 128)
    ins = [x, w, b.reshape(1, m), awl, awr, ab.reshape(1, 1)]
    specs = [pl.BlockSpec((nb, kdim), lambda i: (i, 0)),
             pl.BlockSpec((kdim, m), lambda i: (0, 0)),
             pl.BlockSpec((1, m), lambda i: (0, 0)),
             pl.BlockSpec((m, 1), lambda i: (0, 0)),
             pl.BlockSpec((m, 1), lambda i: (0, 0)),
             pl.BlockSpec((1, 1), lambda i: (0, 0))]
    if den is not None:
        ins.append(den)
        specs.append(pl.BlockSpec((nb, 128), lambda i: (i, 0)))

    def body(x_ref, w_ref, b_ref, awl_ref, awr_ref, ab_ref, *rest):
        t_ref, tab_ref, mx_ref = rest[-3:]
        d = rest[0][...] if den is not None else None
        i = pl.program_id(0)
        xb = _pre_act(x_ref[...], pre, d)
        t = jnp.dot(xb, w_ref[...], preferred_element_type=F32) + b_ref[...]
        tl = jnp.dot(t, awl_ref[...], preferred_element_type=F32) + ab_ref[...]
        tr = jnp.dot(t, awr_ref[...], preferred_element_type=F32)
        if mo > m:
            t = jnp.concatenate(
                [t, jnp.zeros((t.shape[0], mo - m), F32)], axis=1)
        pad = jnp.zeros((xb.shape[0], 127), F32)
        t_ref[...] = jnp.concatenate([t, tl, pad], axis=1)
        tab_ref[...] = jnp.concatenate([tr, pad], axis=1)
        cur = jnp.concatenate(
            [jnp.max(tl, axis=0, keepdims=True),
             jnp.max(tr, axis=0, keepdims=True),
             jnp.full((1, 14), -BIG, F32)], axis=1)

        @pl.when(i == 0)
        def _():
            mx_ref[...] = cur

        @pl.when(i > 0)
        def _():
            mx_ref[...] = jnp.maximum(mx_ref[...], cur)

    return pl.pallas_call(
        body, grid=(n // nb,), in_specs=specs,
        out_specs=[
            pl.BlockSpec((nb, mo + 128), lambda i: (i, 0)),
            pl.BlockSpec((nb, 128), lambda i: (i, 0)),
            pl.BlockSpec((1, 16), lambda i: (0, 0)),
        ],
        out_shape=[
            jax.ShapeDtypeStruct((n, mo + 128), F32),
            jax.ShapeDtypeStruct((n, 128), F32),
            jax.ShapeDtypeStruct((1, 16), F32),
        ])(*ins)


def _tc_ga_expay(tg, b, mx, m, eb=4096):
    e = b.shape[0]
    nch = m // 128
    stripe = nch

    def body(a_ref, b_ref, mx_ref, t_ref, ex_ref, *pouts):
        mxv = mx_ref[...]
        s = mxv[:, 0:1] + mxv[:, 1:2]
        s = jnp.maximum(s, 0.2 * s)
        ee = a_ref[..., 0:1] + b_ref[..., 0:1]
        ee = jnp.maximum(ee, 0.2 * ee)
        ex = jnp.exp(ee - s)
        ex_ref[...] = jnp.concatenate(
            [ex, jnp.zeros((ex.shape[0], 127), F32)], axis=1)
        for j in range(nch):
            pouts[j][...] = ex * t_ref[..., j * 128:(j + 1) * 128]

    return pl.pallas_call(
        body, grid=(e // eb,),
        in_specs=[
            pl.BlockSpec((eb, 128), lambda i: (i, stripe)),
            pl.BlockSpec((eb, 128), lambda i: (i, 0)),
            pl.BlockSpec((1, 16), lambda i: (0, 0)),
            pl.BlockSpec((eb, m), lambda i: (i, 0)),
        ],
        out_specs=[pl.BlockSpec((eb, 128), lambda i: (i, 0))] * (1 + nch),
        out_shape=[jax.ShapeDtypeStruct((e, 128), F32)] * (1 + nch))(
            tg, b, mx, tg)


def _tc_readout(raw, den, pw, pb, awg, awh, ab, gwi, gwh, gbi, gbh, d):
    n = raw.shape[0]

    def body(r_ref, d_ref, pw_ref, pb_ref, ag_ref, ah_ref, ab_ref,
             wi_ref, wh_ref, bi_ref, bh_ref, out_ref):
        h = r_ref[..., 0:d] / (d_ref[:, 0:1] + EPS)
        hv = jnp.dot(h, pw_ref[...], preferred_element_type=F32) + pb_ref[...]
        lg_h = jnp.dot(h, ah_ref[...], preferred_element_type=F32)
        g = jnp.sum(h, axis=0, keepdims=True)
        for _ in range(3):
            gl = jnp.dot(g, ag_ref[...], preferred_element_type=F32) + ab_ref[...]
            logits = lg_h + gl
            logits = jnp.maximum(logits, 0.01 * logits)
            mxl = jnp.max(logits, axis=0, keepdims=True)
            aa = jnp.exp(logits - mxl)
            aa = aa / jnp.sum(aa, axis=0, keepdims=True)
            context = jnp.sum(aa * hv, axis=0, keepdims=True)
            context = jnp.where(context > 0, context,
                                jnp.exp(jnp.minimum(context, 0.0)) - 1.0)
            gi = jnp.dot(context, wi_ref[...], preferred_element_type=F32) + bi_ref[...]
            gh = jnp.dot(g, wh_ref[...], preferred_element_type=F32) + bh_ref[...]
            rr = jax.nn.sigmoid(gi[:, 0:d] + gh[:, 0:d])
            zz = jax.nn.sigmoid(gi[:, d:2 * d] + gh[:, d:2 * d])
            nn_ = jnp.tanh(gi[:, 2 * d:3 * d] + rr * gh[:, 2 * d:3 * d])
            g = (1.0 - zz) * nn_ + zz * g
        out_ref[...] = g

    return pl.pallas_call(
        body,
        out_shape=jax.ShapeDtypeStruct((1, d), F32))(
            raw, den, pw, pb.reshape(1, d), awg, awh, ab.reshape(1, 1),
            gwi, gwh, gbi.reshape(1, 3 * d), gbh.reshape(1, 3 * d))


def kernel(x, edge_index, edge_attr, params):
    p = params
    n, d_in = x.shape
    e = edge_index.shape[1]
    nhalf = n // 2
    heads, hid = p['gat1_al'].shape
    d_out = p['gat3_al'].shape[1]
    src = edge_index[0]
    dst = edge_index[1]

    dsh = _tc_dstsel(dst.reshape(1, e), n).reshape(2 * e)

    h = _tc_linear(x, p['proj_W'], p['proj_b'], act="relu", out_w=128)
    u = _tc_linear(edge_attr, p['enet_W1'], p['enet_b1'], act="relu", nb=2048)
    kk = u.shape[1]
    w2r = p['enet_W2'].reshape(kk, d_in, d_in)
    w2r = jnp.concatenate(
        [w2r, jnp.zeros((kk, 128 - d_in, d_in), F32)], axis=1)
    b2r = _pad_rows(p['enet_b2'].reshape(d_in, d_in), 128)
    gru_wi = _pad_rows(p['gru_Wi'], 128)
    gru_wh = _pad_rows(p['gru_Wh'], 128)
    hidden = h
    for _ in range(3):
        (hs,) = _sc_gather([h], [src])
        msg = _mp_messages(u, hs, w2r, b2r)
        (agg,) = _sc_scatter([msg], dsh, n)
        h = _tc_gru(agg, hidden, gru_wi, gru_wh, p['gru_bi'], p['gru_bh'],
                    d_in)
        hidden = h

    hcur = h
    for nm, dh, pre, wpad in (("gat1", hid, "none", 128),
                              ("gat2", hid, "elu", 0),
                              ("gat3", d_out, "elu", 0)):
        w = _pad_rows(p[nm + '_W'], wpad) if wpad else p[nm + '_W']
        zt, tabr, mx = _tc_gat_z(hcur, w, p[nm + '_al'], p[nm + '_ar'],
                                 heads, dh, pre=pre)
        zg, b_g = _sc_gather([zt, tabr], [src, dst])
        ex = _tc_gat_ex(zg, b_g, mx, heads, heads * dh)
        (den,) = _sc_scatter([ex], dsh, n)
        (bg,) = _sc_gather([den], [dst])
        pays = _tc_gat_pay(zg, ex, bg, heads, dh)
        outs = _sc_scatter(list(pays), dsh, n)
        hcur = outs[0] if len(outs) == 1 else jnp.concatenate(outs, axis=1)

    raw, den_prev = hcur, None
    for nm, wpad in (("ga1", 128), ("ga2", 0), ("ga3", 0)):
        aw = p[nm + '_aw']
        m = aw.shape[0] // 2
        w = _pad_rows(p[nm + '_W'], wpad) if wpad else p[nm + '_W']
        t, tabr, mx = _tc_ga_t(raw, w, p[nm + '_b'], aw[:m], aw[m:],
                               p[nm + '_ab'][0], den=den_prev)
        tg, b_g = _sc_gather([t, tabr], [src, dst])
        expays = _tc_ga_expay(tg, b_g, mx, max(p[nm + '_W'].shape[1], 128))
        (den_prev,) = _sc_scatter([expays[0]], dsh, n)
        souts = _sc_scatter(list(expays[1:]), dsh, n)
        raw = (souts[0] if len(souts) == 1
               else jnp.concatenate(souts, axis=1))

    attw = p['ro_attW']
    return _tc_readout(raw, den_prev, p['ro_pW'], p['ro_pb'],
                       attw[:d_out], attw[d_out:], p['ro_attb'][0],
                       p['ro_gWi'], p['ro_gWh'], p['ro_gbi'], p['ro_gbh'],
                       d_out)
